# baseline (speedup 1.0000x reference)
import functools

import jax
import jax.numpy as jnp
from jax import lax
from jax.experimental import pallas as pl
from jax.experimental.pallas import tpu as pltpu
from jax.experimental.pallas import tpu_sc as plsc

B = 16384
L = 200
LP = 208
E = 64
NC = 2
NS = 16
NW = NC * NS
RPW = B // NW
D1, D2 = 256, 128
BT = 1024


BR = 64
NSLOT = 3
EP = 2 * E

_DN = lax.GatherDimensionNumbers(offset_dims=(), collapsed_slice_dims=(0,),
                                 start_index_map=(0,))


def _lane_total(v):
  lanes = lax.iota(jnp.int32, 16)
  for shift in (8, 4, 2, 1):
    perm = jnp.bitwise_xor(lanes, shift)
    v = v + lax.gather(v, perm[:, None], _DN, (1,),
                       mode=lax.GatherScatterMode.PROMISE_IN_BOUNDS)
  return v


def _sc_features(xseq, i_uid, i_gen, i_city, i_iid, i_cate,
                 t_uid, t_gen, t_city, t_iid, t_cate,
                 o_uid, o_gen, o_city, o_iid, o_cate, o_seq,
                 idxf_v, pidx_v, parf_v, idxblk_v, rows_v, sbuf_v, e0_v,
                 semf, sem0, sem1, sem2):
  wid = lax.axis_index("s") * NC + lax.axis_index("c")
  base = wid * RPW

  for idx_hbm, paired, tab, out in (
      (i_uid, True, t_uid, o_uid), (i_gen, True, t_gen, o_gen),
      (i_city, True, t_city, o_city), (i_iid, False, t_iid, o_iid),
      (i_cate, True, t_cate, o_cate)):
    for fb in range(RPW // BR):
      fbase = base + fb * BR
      pltpu.sync_copy(idx_hbm.at[pl.ds(fbase, BR)], idxf_v)
      for k in range(BR // 16):
        ch = idxf_v[pl.ds(k * 16, 16)]
        if paired:
          pidx_v[pl.ds(k * 16, 16)] = lax.shift_right_logical(ch, 1)
          parf_v[k] = jnp.bitwise_and(ch, 1)
        else:
          pidx_v[pl.ds(k * 16, 16)] = ch
      fcopy = pltpu.make_async_copy(tab.at[pidx_v], sbuf_v.at[0, pl.ds(0, BR)],
                                    semf)
      fcopy.start()
      fcopy.wait()

      def fsel(r, carry):
        if paired:
          pchunk = parf_v[r >> 4]
          psplat = lax.gather(
              pchunk, jnp.full((16, 1), r & 15, jnp.int32), _DN, (1,),
              mode=lax.GatherScatterMode.PROMISE_IN_BOUNDS)
          pf = psplat.astype(jnp.float32)
        for c in range(4):
          val = sbuf_v[0, r, pl.ds(c * 16, 16)]
          if paired:
            hi = sbuf_v[0, r, pl.ds(E + c * 16, 16)]
            val = val + pf * (hi - val)
          rows_v[r, pl.ds(c * 16, 16)] = val
        return carry

      lax.fori_loop(0, BR, fsel, jnp.int32(0))
      pltpu.sync_copy(rows_v, out.at[pl.ds(fbase, BR)])

  pltpu.sync_copy(t_iid.at[0], e0_v)
  sems = (sem0, sem1, sem2)

  def _gathers(r, s):
    return ((t_iid.at[idxblk_v.at[r, pl.ds(0, 128)]],
             sbuf_v.at[s, pl.ds(0, 128)], sems[s]),
            (t_iid.at[idxblk_v.at[r, pl.ds(128, LP - 128)]],
             sbuf_v.at[s, pl.ds(128, LP - 128)], sems[s]))

  def issue(r, s):
    for src, dst, sem in _gathers(r, s):
      pltpu.make_async_copy(src, dst, sem).start()

  def drain(r, s):
    for src, dst, sem in _gathers(r, s):
      pltpu.make_async_copy(src, dst, sem).wait()

  def compute(r, s):
    nzi = jnp.zeros((16,), jnp.int32)
    for c in range(LP // 16):
      ch = idxblk_v[r, pl.ds(c * 16, 16)]
      nzi = nzi + (1 - jnp.minimum(ch, 1))
    n0v = _lane_total(nzi.astype(jnp.float32))
    rcp = jnp.float32(1.0) / (jnp.float32(LP) - n0v + jnp.float32(1e-8))

    def sbody(j, accs):
      out = list(accs)
      for u in range(8):
        row = j * 8 + u
        for c in range(4):
          out[c] = out[c] + sbuf_v[s, row, pl.ds(c * 16, 16)]
      return tuple(out)

    accs = lax.fori_loop(0, LP // 8, sbody,
                         tuple(jnp.zeros((16,), jnp.float32) for _ in range(4)))
    for c in range(4):
      avg = (accs[c] - n0v * e0_v[pl.ds(c * 16, 16)]) * rcp
      rows_v[r, pl.ds(c * 16, 16)] = avg

  for blk in range(RPW // BR):
    pltpu.sync_copy(xseq.at[pl.ds(base + blk * BR, BR)], idxblk_v)
    for s in range(NSLOT):
      issue(s, s)

    def group(g, carry):
      r = g * NSLOT
      for s in range(NSLOT):
        drain(r + s, s)
        compute(r + s, s)

        @pl.when(r + s + NSLOT < BR)
        def _():
          issue(r + s + NSLOT, s)

      return carry

    lax.fori_loop(0, BR // NSLOT, group, jnp.int32(0))
    for s in range((BR // NSLOT) * NSLOT, BR):
      drain(s, s % NSLOT)
      compute(s, s % NSLOT)
    pltpu.sync_copy(rows_v, o_seq.at[pl.ds(base + blk * BR, BR)])


_sc_embed = functools.partial(
    pl.kernel,
    out_type=[jax.ShapeDtypeStruct((B, E), jnp.float32)] * 6,
    mesh=plsc.VectorSubcoreMesh(core_axis_name="c", subcore_axis_name="s"),
    scratch_types=[
        pltpu.VMEM((BR,), jnp.int32),
        pltpu.VMEM((BR,), jnp.int32),
        pltpu.VMEM((BR // 16, 16), jnp.int32),
        pltpu.VMEM((BR, LP), jnp.int32),
        pltpu.VMEM((BR, E), jnp.float32),
        pltpu.VMEM((NSLOT, LP, EP), jnp.float32),
        pltpu.VMEM((EP,), jnp.float32),
        pltpu.SemaphoreType.DMA,
        pltpu.SemaphoreType.DMA,
        pltpu.SemaphoreType.DMA,
        pltpu.SemaphoreType.DMA,
    ],
)(_sc_features)


def _l1_body(f0, f1, f2, f3, f4, f5, xsc, w, wsc, b, h_ref, s_ref, q_ref):
  hid = jnp.concatenate(
      [f0[...], f1[...], f2[...], f3[...], f4[...], f5[...]], axis=1)
  h = jnp.dot(hid, w[...], preferred_element_type=jnp.float32)
  xv = xsc[...]
  wv = wsc[...]
  h = h + xv[:, 0:1] * wv[0:1, :] + xv[:, 1:2] * wv[1:2, :] + b[...]
  h_ref[...] = h

  @pl.when(pl.program_id(0) == 0)
  def _():
    s_ref[...] = jnp.zeros_like(s_ref)
    q_ref[...] = jnp.zeros_like(q_ref)

  s_ref[...] += jnp.sum(h, axis=0, keepdims=True)
  q_ref[...] += jnp.sum(h * h, axis=0, keepdims=True)


def _l2_body(h0, sc, sh, w, b, h_ref, s_ref, q_ref):
  a = jnp.maximum(h0[...] * sc[...] + sh[...], 0.0)
  h = jnp.dot(a, w[...], preferred_element_type=jnp.float32) + b[...]
  h_ref[...] = h

  @pl.when(pl.program_id(0) == 0)
  def _():
    s_ref[...] = jnp.zeros_like(s_ref)
    q_ref[...] = jnp.zeros_like(q_ref)

  s_ref[...] += jnp.sum(h, axis=0, keepdims=True)
  q_ref[...] += jnp.sum(h * h, axis=0, keepdims=True)


def _l3_body(h1, sc, sh, w, b, o_ref):
  a = jnp.maximum(h1[...] * sc[...] + sh[...], 0.0)
  o_ref[...] = jnp.dot(a, w[...], preferred_element_type=jnp.float32) + b[...]


def _full(shape):
  return pl.BlockSpec(shape, lambda i: (0, 0))


def _tile(width):
  return pl.BlockSpec((BT, width), lambda i: (i, 0))


def _layer1(feats, xsc, w, wsc, b):
  return pl.pallas_call(
      _l1_body,
      grid=(B // BT,),
      in_specs=[_tile(E)] * 6 + [_tile(2), _full((6 * E, 2 * D1)),
                                 _full((2, 2 * D1)), _full((1, 2 * D1))],
      out_specs=[_tile(2 * D1), _full((1, 2 * D1)), _full((1, 2 * D1))],
      out_shape=[jax.ShapeDtypeStruct((B, 2 * D1), jnp.float32),
                 jax.ShapeDtypeStruct((1, 2 * D1), jnp.float32),
                 jax.ShapeDtypeStruct((1, 2 * D1), jnp.float32)],
  )(*feats, xsc, w, wsc, b)


def _layer2(h0, sc, sh, w, b):
  return pl.pallas_call(
      _l2_body,
      grid=(B // BT,),
      in_specs=[_tile(2 * D1), _full((1, 2 * D1)), _full((1, 2 * D1)),
                _full((2 * D1, 2 * D2)), _full((1, 2 * D2))],
      out_specs=[_tile(2 * D2), _full((1, 2 * D2)), _full((1, 2 * D2))],
      out_shape=[jax.ShapeDtypeStruct((B, 2 * D2), jnp.float32),
                 jax.ShapeDtypeStruct((1, 2 * D2), jnp.float32),
                 jax.ShapeDtypeStruct((1, 2 * D2), jnp.float32)],
  )(h0, sc, sh, w, b)


def _layer3(h1, sc, sh, w, b):
  return pl.pallas_call(
      _l3_body,
      grid=(B // BT,),
      in_specs=[_tile(2 * D2), _full((1, 2 * D2)), _full((1, 2 * D2)),
                _full((2 * D2, 2)), _full((1, 2))],
      out_specs=_tile(2),
      out_shape=jax.ShapeDtypeStruct((B, 2), jnp.float32),
  )(h1, sc, sh, w, b)


def _bn_fold(s, q, g, be):
  mu = s / B
  var = q / B - mu * mu
  scale = g / jnp.sqrt(var + 1e-5)
  return scale, be - mu * scale


def kernel(x, x_seq, emb_user_id, emb_user_gender, emb_user_city, emb_item_id,
           emb_item_cate,
           t0_W0, t0_b0, t0_g0, t0_be0, t0_W1, t0_b1, t0_g1, t0_be1,
           t0_Wout, t0_bout,
           t1_W0, t1_b0, t1_g0, t1_be0, t1_W1, t1_b1, t1_g1, t1_be1,
           t1_Wout, t1_bout):
  xi = x.astype(jnp.int32)
  xseq_p = jnp.pad(x_seq.astype(jnp.int32), ((0, 0), (0, LP - L)))

  t_iid2 = jnp.concatenate([emb_item_id, emb_item_id], axis=1)
  t_uid2 = emb_user_id.reshape(-1, 2 * E)
  t_gen2 = jnp.pad(emb_user_gender, ((0, 1), (0, 0))).reshape(-1, 2 * E)
  t_city2 = emb_user_city.reshape(-1, 2 * E)
  t_cate2 = emb_item_cate.reshape(-1, 2 * E)

  feats = _sc_embed(xseq_p, xi[:, 0], xi[:, 2], xi[:, 3], xi[:, 4], xi[:, 5],
                    t_uid2, t_gen2, t_city2, t_iid2, t_cate2)
  xsc = jnp.stack([x[:, 1], x[:, 6]], axis=1)

  def _perm(W):
    We = jnp.concatenate([W[0:64], W[65:129], W[129:193], W[193:257],
                          W[257:321], W[322:386]], axis=0)
    return We, jnp.stack([W[64], W[321]], axis=0)

  W0e0, Wsc0 = _perm(t0_W0)
  W0e1, Wsc1 = _perm(t1_W0)
  W0cat = jnp.concatenate([W0e0, W0e1], axis=1)
  Wsccat = jnp.concatenate([Wsc0, Wsc1], axis=1)
  b0cat = jnp.concatenate([t0_b0, t1_b0])[None, :]
  g0cat = jnp.concatenate([t0_g0, t1_g0])[None, :]
  be0cat = jnp.concatenate([t0_be0, t1_be0])[None, :]
  W1bd = (jnp.zeros((2 * D1, 2 * D2), jnp.float32)
          .at[:D1, :D2].set(t0_W1).at[D1:, D2:].set(t1_W1))
  b1cat = jnp.concatenate([t0_b1, t1_b1])[None, :]
  g1cat = jnp.concatenate([t0_g1, t1_g1])[None, :]
  be1cat = jnp.concatenate([t0_be1, t1_be1])[None, :]
  Woutbd = (jnp.zeros((2 * D2, 2), jnp.float32)
            .at[:D2, 0:1].set(t0_Wout).at[D2:, 1:2].set(t1_Wout))
  boutcat = jnp.concatenate([t0_bout, t1_bout])[None, :]

  h0, s0, q0 = _layer1(feats, xsc, W0cat, Wsccat, b0cat)
  sc0, sh0 = _bn_fold(s0, q0, g0cat, be0cat)
  h1, s1, q1 = _layer2(h0, sc0, sh0, W1bd, b1cat)
  sc1, sh1 = _bn_fold(s1, q1, g1cat, be1cat)
  out = _layer3(h1, sc1, sh1, Woutbd, boutcat)
  return (out[:, 0:1], out[:, 1:2])


# speedup vs baseline: 1.4929x; 1.4929x over previous
import functools

import jax
import jax.numpy as jnp
from jax import lax
from jax.experimental import pallas as pl
from jax.experimental.pallas import tpu as pltpu
from jax.experimental.pallas import tpu_sc as plsc

B = 16384
L = 200
LP = 208
E = 64
NC = 2
NS = 16
NW = NC * NS
RPW = B // NW
D1, D2 = 256, 128
BT = 1024

BR = 128
NSLOT = 4

_DN = lax.GatherDimensionNumbers(offset_dims=(), collapsed_slice_dims=(0,),
                                 start_index_map=(0,))

_UNPACK_PERM = ([2 * i for i in range(16)] + [2 * i + 1 for i in range(16)]
                + [32 + 2 * i for i in range(16)]
                + [32 + 2 * i + 1 for i in range(16)])


def _bf16_unpack(wi):
  even = lax.bitcast_convert_type(lax.shift_left(wi, 16), jnp.float32)
  odd = lax.bitcast_convert_type(jnp.bitwise_and(wi, jnp.int32(-65536)),
                                 jnp.float32)
  return even, odd


def _lane_total(v):
  lanes = lax.iota(jnp.int32, 16)
  for shift in (8, 4, 2, 1):
    perm = jnp.bitwise_xor(lanes, shift)
    v = v + lax.gather(v, perm[:, None], _DN, (1,),
                       mode=lax.GatherScatterMode.PROMISE_IN_BOUNDS)
  return v


def _sc_features(xseq, i_uid, i_gen, i_city, i_iid, i_cate,
                 t_uid, t_gen, t_city, t_iid, t_cate,
                 o_uid, o_gen, o_city, o_iid, o_cate, o_seq,
                 idxf_v, fbuf_v, idxblk_v, rows_v, sbuf_v, e0_v,
                 semf, sem0, sem1, sem2, sem3):
  wid = lax.axis_index("s") * NC + lax.axis_index("c")
  base = wid * RPW

  for idx_hbm, tab, out in ((i_uid, t_uid, o_uid), (i_gen, t_gen, o_gen),
                            (i_city, t_city, o_city), (i_iid, t_iid, o_iid),
                            (i_cate, t_cate, o_cate)):
    pltpu.sync_copy(idx_hbm.at[pl.ds(base, RPW)], idxf_v)
    for c in range(RPW // 128):
      pltpu.make_async_copy(tab.at[idxf_v.at[pl.ds(c * 128, 128)]],
                            fbuf_v.at[pl.ds(c * 128, 128)], semf).start()
    for c in range(RPW // 128):
      pltpu.make_async_copy(tab.at[idxf_v.at[pl.ds(c * 128, 128)]],
                            fbuf_v.at[pl.ds(c * 128, 128)], semf).wait()
    pltpu.sync_copy(fbuf_v, out.at[pl.ds(base, RPW)])

  pltpu.sync_copy(t_iid.at[0], e0_v)
  e0a0, e0b0 = _bf16_unpack(e0_v[pl.ds(0, 16)])
  e0a1, e0b1 = _bf16_unpack(e0_v[pl.ds(16, 16)])
  e0ch = (e0a0, e0b0, e0a1, e0b1)
  sems = (sem0, sem1, sem2, sem3)

  def _gathers(r, s):
    return ((t_iid.at[idxblk_v.at[r, pl.ds(0, 128)]],
             sbuf_v.at[s, pl.ds(0, 128)], sems[s]),
            (t_iid.at[idxblk_v.at[r, pl.ds(128, LP - 128)]],
             sbuf_v.at[s, pl.ds(128, LP - 128)], sems[s]))

  def issue(r, s):
    for src, dst, sem in _gathers(r, s):
      pltpu.make_async_copy(src, dst, sem).start()

  def drain(r, s):
    for src, dst, sem in _gathers(r, s):
      pltpu.make_async_copy(src, dst, sem).wait()

  def compute(r, s):
    nzi = jnp.zeros((16,), jnp.int32)
    for c in range(LP // 16):
      ch = idxblk_v[r, pl.ds(c * 16, 16)]
      nzi = nzi + (1 - jnp.minimum(ch, 1))
    n0v = _lane_total(nzi.astype(jnp.float32))
    rcp = jnp.float32(1.0) / (jnp.float32(LP) - n0v + jnp.float32(1e-8))

    def sbody(j, accs):
      out = list(accs)
      for u in range(8):
        row = j * 8 + u
        for h in range(2):
          w = sbuf_v[s, row, pl.ds(h * 16, 16)]
          a, b = _bf16_unpack(w)
          out[2 * h] = out[2 * h] + a
          out[2 * h + 1] = out[2 * h + 1] + b
      return tuple(out)

    accs = lax.fori_loop(0, LP // 8, sbody,
                         tuple(jnp.zeros((16,), jnp.float32) for _ in range(4)))
    for c in range(4):
      avg = (accs[c] - n0v * e0ch[c]) * rcp
      rows_v[r, pl.ds(c * 16, 16)] = avg

  for blk in range(RPW // BR):
    pltpu.sync_copy(xseq.at[pl.ds(base + blk * BR, BR)], idxblk_v)
    for s in range(NSLOT):
      issue(s, s)

    def group(g, carry):
      r = g * NSLOT
      for s in range(NSLOT):
        drain(r + s, s)
        compute(r + s, s)

        @pl.when(r + s + NSLOT < BR)
        def _():
          issue(r + s + NSLOT, s)

      return carry

    lax.fori_loop(0, BR // NSLOT, group, jnp.int32(0))
    pltpu.sync_copy(rows_v, o_seq.at[pl.ds(base + blk * BR, BR)])


_sc_embed = functools.partial(
    pl.kernel,
    out_type=[jax.ShapeDtypeStruct((B, E // 2), jnp.int32)] * 5
    + [jax.ShapeDtypeStruct((B, E), jnp.float32)],
    mesh=plsc.VectorSubcoreMesh(core_axis_name="c", subcore_axis_name="s"),
    compiler_params=pltpu.CompilerParams(use_tc_tiling_on_sc=False),
    scratch_types=[
        pltpu.VMEM((RPW,), jnp.int32),
        pltpu.VMEM((RPW, E // 2), jnp.int32),
        pltpu.VMEM((BR, LP), jnp.int32),
        pltpu.VMEM((BR, E), jnp.float32),
        pltpu.VMEM((NSLOT, LP, E // 2), jnp.int32),
        pltpu.VMEM((E // 2,), jnp.int32),
        pltpu.SemaphoreType.DMA,
        pltpu.SemaphoreType.DMA,
        pltpu.SemaphoreType.DMA,
        pltpu.SemaphoreType.DMA,
        pltpu.SemaphoreType.DMA,
    ],
)(_sc_features)


def _l1_body(f0, f1, f2, f3, f4, f5, xsc, w, wsc, b, h_ref, s_ref, q_ref):
  parts = []
  for f in (f0, f1, f2, f3, f4):
    wi = f[...]
    parts.append(lax.bitcast_convert_type(wi << 16, jnp.float32))
    parts.append(lax.bitcast_convert_type(wi & jnp.int32(-65536), jnp.float32))
  parts.append(f5[...])
  hid = jnp.concatenate(parts, axis=1)
  h = jnp.dot(hid, w[...], preferred_element_type=jnp.float32,
              precision=lax.Precision.HIGHEST)
  xv = xsc[...]
  wv = wsc[...]
  h = h + xv[:, 0:1] * wv[0:1, :] + xv[:, 1:2] * wv[1:2, :] + b[...]
  h_ref[...] = h

  @pl.when(pl.program_id(0) == 0)
  def _():
    s_ref[...] = jnp.zeros_like(s_ref)
    q_ref[...] = jnp.zeros_like(q_ref)

  s_ref[...] += jnp.sum(h, axis=0, keepdims=True)
  q_ref[...] += jnp.sum(h * h, axis=0, keepdims=True)


def _l2_body(h0, sc, sh, w, b, h_ref, s_ref, q_ref):
  a = jnp.maximum(h0[...] * sc[...] + sh[...], 0.0)
  h = jnp.dot(a, w[...], preferred_element_type=jnp.float32,
              precision=lax.Precision.HIGHEST) + b[...]
  h_ref[...] = h

  @pl.when(pl.program_id(0) == 0)
  def _():
    s_ref[...] = jnp.zeros_like(s_ref)
    q_ref[...] = jnp.zeros_like(q_ref)

  s_ref[...] += jnp.sum(h, axis=0, keepdims=True)
  q_ref[...] += jnp.sum(h * h, axis=0, keepdims=True)


def _l3_body(h1, sc, sh, w, b, o_ref):
  a = jnp.maximum(h1[...] * sc[...] + sh[...], 0.0)
  o_ref[...] = jnp.dot(a, w[...], preferred_element_type=jnp.float32,
                       precision=lax.Precision.HIGHEST) + b[...]


def _full(shape):
  return pl.BlockSpec(shape, lambda i: (0, 0))


def _tile(width):
  return pl.BlockSpec((BT, width), lambda i: (i, 0))


def _layer1(feats, xsc, w, wsc, b):
  return pl.pallas_call(
      _l1_body,
      grid=(B // BT,),
      in_specs=[_tile(E // 2)] * 5 + [_tile(E)] + [_tile(2), _full((6 * E, 2 * D1)),
                                 _full((2, 2 * D1)), _full((1, 2 * D1))],
      out_specs=[_tile(2 * D1), _full((1, 2 * D1)), _full((1, 2 * D1))],
      out_shape=[jax.ShapeDtypeStruct((B, 2 * D1), jnp.float32),
                 jax.ShapeDtypeStruct((1, 2 * D1), jnp.float32),
                 jax.ShapeDtypeStruct((1, 2 * D1), jnp.float32)],
  )(*feats, xsc, w, wsc, b)


def _layer2(h0, sc, sh, w, b):
  return pl.pallas_call(
      _l2_body,
      grid=(B // BT,),
      in_specs=[_tile(2 * D1), _full((1, 2 * D1)), _full((1, 2 * D1)),
                _full((2 * D1, 2 * D2)), _full((1, 2 * D2))],
      out_specs=[_tile(2 * D2), _full((1, 2 * D2)), _full((1, 2 * D2))],
      out_shape=[jax.ShapeDtypeStruct((B, 2 * D2), jnp.float32),
                 jax.ShapeDtypeStruct((1, 2 * D2), jnp.float32),
                 jax.ShapeDtypeStruct((1, 2 * D2), jnp.float32)],
  )(h0, sc, sh, w, b)


def _layer3(h1, sc, sh, w, b):
  return pl.pallas_call(
      _l3_body,
      grid=(B // BT,),
      in_specs=[_tile(2 * D2), _full((1, 2 * D2)), _full((1, 2 * D2)),
                _full((2 * D2, 2)), _full((1, 2))],
      out_specs=_tile(2),
      out_shape=jax.ShapeDtypeStruct((B, 2), jnp.float32),
  )(h1, sc, sh, w, b)


def _bn_fold(s, q, g, be):
  mu = s / B
  var = q / B - mu * mu
  scale = g / jnp.sqrt(var + 1e-5)
  return scale, be - mu * scale


def kernel(x, x_seq, emb_user_id, emb_user_gender, emb_user_city, emb_item_id,
           emb_item_cate,
           t0_W0, t0_b0, t0_g0, t0_be0, t0_W1, t0_b1, t0_g1, t0_be1,
           t0_Wout, t0_bout,
           t1_W0, t1_b0, t1_g0, t1_be0, t1_W1, t1_b1, t1_g1, t1_be1,
           t1_Wout, t1_bout):
  xi = x.astype(jnp.int32)
  xseq_p = jnp.pad(x_seq.astype(jnp.int32), ((0, 0), (0, LP - L)))
  def _pack(t):
    tb = t.astype(jnp.bfloat16).reshape(t.shape[0], E // 2, 2)
    return lax.bitcast_convert_type(tb, jnp.int32)

  tb_uid = _pack(emb_user_id)
  tb_gen = _pack(emb_user_gender)
  tb_city = _pack(emb_user_city)
  tb_iid = _pack(emb_item_id)
  tb_cate = _pack(emb_item_cate)
  feats = _sc_embed(xseq_p, xi[:, 0], xi[:, 2], xi[:, 3], xi[:, 4], xi[:, 5],
                    tb_uid, tb_gen, tb_city, tb_iid, tb_cate)
  xsc = jnp.stack([x[:, 1], x[:, 6]], axis=1)

  perm = jnp.array(_UNPACK_PERM, jnp.int32)
  fperm = jnp.array([2 * i for i in range(32)] + [2 * i + 1 for i in range(32)],
                    jnp.int32)

  def _permW(W):
    We = jnp.concatenate([W[0:64][fperm], W[65:129][fperm], W[129:193][fperm],
                          W[193:257][fperm], W[257:321][fperm],
                          W[322:386][perm]], axis=0)
    return We, jnp.stack([W[64], W[321]], axis=0)

  W0e0, Wsc0 = _permW(t0_W0)
  W0e1, Wsc1 = _permW(t1_W0)
  W0cat = jnp.concatenate([W0e0, W0e1], axis=1)
  Wsccat = jnp.concatenate([Wsc0, Wsc1], axis=1)
  b0cat = jnp.concatenate([t0_b0, t1_b0])[None, :]
  g0cat = jnp.concatenate([t0_g0, t1_g0])[None, :]
  be0cat = jnp.concatenate([t0_be0, t1_be0])[None, :]
  W1bd = (jnp.zeros((2 * D1, 2 * D2), jnp.float32)
          .at[:D1, :D2].set(t0_W1).at[D1:, D2:].set(t1_W1))
  b1cat = jnp.concatenate([t0_b1, t1_b1])[None, :]
  g1cat = jnp.concatenate([t0_g1, t1_g1])[None, :]
  be1cat = jnp.concatenate([t0_be1, t1_be1])[None, :]
  Woutbd = (jnp.zeros((2 * D2, 2), jnp.float32)
            .at[:D2, 0:1].set(t0_Wout).at[D2:, 1:2].set(t1_Wout))
  boutcat = jnp.concatenate([t0_bout, t1_bout])[None, :]

  h0, s0, q0 = _layer1(feats, xsc, W0cat, Wsccat, b0cat)
  sc0, sh0 = _bn_fold(s0, q0, g0cat, be0cat)
  h1, s1, q1 = _layer2(h0, sc0, sh0, W1bd, b1cat)
  sc1, sh1 = _bn_fold(s1, q1, g1cat, be1cat)
  out = _layer3(h1, sc1, sh1, Woutbd, boutcat)
  return (out[:, 0:1], out[:, 1:2])


# speedup vs baseline: 1.6031x; 1.0738x over previous
import functools

import jax
import jax.numpy as jnp
from jax import lax
from jax.experimental import pallas as pl
from jax.experimental.pallas import tpu as pltpu
from jax.experimental.pallas import tpu_sc as plsc

B = 16384
L = 200
LP = 208
E = 64
NC = 2
NS = 16
NW = NC * NS
RPW = B // NW
D1, D2 = 256, 128
BT = 1024

BR = 128
NSLOT = 4

_DN = lax.GatherDimensionNumbers(offset_dims=(), collapsed_slice_dims=(0,),
                                 start_index_map=(0,))

_UNPACK_PERM = (list(range(0, 16)) + list(range(32, 48))
                + list(range(16, 32)) + list(range(48, 64)))


def _bf16_unpack(wi):
  even = lax.bitcast_convert_type(lax.shift_left(wi, 16), jnp.float32)
  odd = lax.bitcast_convert_type(jnp.bitwise_and(wi, jnp.int32(-65536)),
                                 jnp.float32)
  return even, odd


def _lane_total(v):
  lanes = lax.iota(jnp.int32, 16)
  for shift in (8, 4, 2, 1):
    perm = jnp.bitwise_xor(lanes, shift)
    v = v + lax.gather(v, perm[:, None], _DN, (1,),
                       mode=lax.GatherScatterMode.PROMISE_IN_BOUNDS)
  return v


def _sc_features(xseq, i_uid, i_gen, i_city, i_iid, i_cate,
                 t_uid, t_gen, t_city, t_iid, t_cate,
                 o_uid, o_gen, o_city, o_iid, o_cate, o_seq,
                 idxf_v, fbuf_v, idxblk_v, rows_v, sbuf_v, e0_v,
                 semf, sem0, sem1, sem2, sem3):
  wid = lax.axis_index("s") * NC + lax.axis_index("c")
  base = wid * RPW

  for idx_hbm, tab, out in ((i_uid, t_uid, o_uid), (i_gen, t_gen, o_gen),
                            (i_city, t_city, o_city), (i_iid, t_iid, o_iid),
                            (i_cate, t_cate, o_cate)):
    pltpu.sync_copy(idx_hbm.at[pl.ds(base, RPW)], idxf_v)
    for c in range(RPW // 128):
      pltpu.make_async_copy(tab.at[idxf_v.at[pl.ds(c * 128, 128)]],
                            fbuf_v.at[pl.ds(c * 128, 128)], semf).start()
    for c in range(RPW // 128):
      pltpu.make_async_copy(tab.at[idxf_v.at[pl.ds(c * 128, 128)]],
                            fbuf_v.at[pl.ds(c * 128, 128)], semf).wait()
    pltpu.sync_copy(fbuf_v, out.at[pl.ds(base, RPW)])

  pltpu.sync_copy(t_iid.at[0], e0_v)
  e0a0, e0b0 = _bf16_unpack(e0_v[pl.ds(0, 16)])
  e0a1, e0b1 = _bf16_unpack(e0_v[pl.ds(16, 16)])
  e0ch = (e0a0, e0b0, e0a1, e0b1)
  sems = (sem0, sem1, sem2, sem3)

  def _gathers(r, s):
    return ((t_iid.at[idxblk_v.at[r, pl.ds(0, 128)]],
             sbuf_v.at[s, pl.ds(0, 128)], sems[s]),
            (t_iid.at[idxblk_v.at[r, pl.ds(128, LP - 128)]],
             sbuf_v.at[s, pl.ds(128, LP - 128)], sems[s]))

  def issue(r, s):
    for src, dst, sem in _gathers(r, s):
      pltpu.make_async_copy(src, dst, sem).start()

  def drain(r, s):
    for src, dst, sem in _gathers(r, s):
      pltpu.make_async_copy(src, dst, sem).wait()

  def compute(r, s):
    nzi = jnp.zeros((16,), jnp.int32)
    for c in range(LP // 16):
      ch = idxblk_v[r, pl.ds(c * 16, 16)]
      nzi = nzi + (1 - jnp.minimum(ch, 1))
    n0v = _lane_total(nzi.astype(jnp.float32))
    rcp = jnp.float32(1.0) / (jnp.float32(LP) - n0v + jnp.float32(1e-8))

    def sbody(j, accs):
      out = list(accs)
      for u in range(8):
        row = j * 8 + u
        for h in range(2):
          w = sbuf_v[s, row, pl.ds(h * 16, 16)]
          a, b = _bf16_unpack(w)
          out[2 * h] = out[2 * h] + a
          out[2 * h + 1] = out[2 * h + 1] + b
      return tuple(out)

    accs = lax.fori_loop(0, LP // 8, sbody,
                         tuple(jnp.zeros((16,), jnp.float32) for _ in range(4)))
    for c in range(4):
      avg = (accs[c] - n0v * e0ch[c]) * rcp
      rows_v[r, pl.ds(c * 16, 16)] = avg

  for blk in range(RPW // BR):
    pltpu.sync_copy(xseq.at[pl.ds(base + blk * BR, BR)], idxblk_v)
    for s in range(NSLOT):
      issue(s, s)

    def group(g, carry):
      r = g * NSLOT
      for s in range(NSLOT):
        drain(r + s, s)
        compute(r + s, s)

        @pl.when(r + s + NSLOT < BR)
        def _():
          issue(r + s + NSLOT, s)

      return carry

    lax.fori_loop(0, BR // NSLOT, group, jnp.int32(0))
    pltpu.sync_copy(rows_v, o_seq.at[pl.ds(base + blk * BR, BR)])


_sc_embed = functools.partial(
    pl.kernel,
    out_type=[jax.ShapeDtypeStruct((B, E // 2), jnp.int32)] * 5
    + [jax.ShapeDtypeStruct((B, E), jnp.float32)],
    mesh=plsc.VectorSubcoreMesh(core_axis_name="c", subcore_axis_name="s"),
    compiler_params=pltpu.CompilerParams(use_tc_tiling_on_sc=False),
    scratch_types=[
        pltpu.VMEM((RPW,), jnp.int32),
        pltpu.VMEM((RPW, E // 2), jnp.int32),
        pltpu.VMEM((BR, LP), jnp.int32),
        pltpu.VMEM((BR, E), jnp.float32),
        pltpu.VMEM((NSLOT, LP, E // 2), jnp.int32),
        pltpu.VMEM((E // 2,), jnp.int32),
        pltpu.SemaphoreType.DMA,
        pltpu.SemaphoreType.DMA,
        pltpu.SemaphoreType.DMA,
        pltpu.SemaphoreType.DMA,
        pltpu.SemaphoreType.DMA,
    ],
)(_sc_features)


def _l1_body(f0, f1, f2, f3, f4, f5, xsc, w, wsc, b, h_ref, s_ref, q_ref):
  parts = []
  for f in (f0, f1, f2, f3, f4):
    wi = f[...]
    parts.append(lax.bitcast_convert_type(wi << 16, jnp.float32))
    parts.append(lax.bitcast_convert_type(wi & jnp.int32(-65536), jnp.float32))
  parts.append(f5[...])
  hid = jnp.concatenate(parts, axis=1)
  h = jnp.dot(hid, w[...], preferred_element_type=jnp.float32,
              precision=lax.Precision.HIGHEST)
  xv = xsc[...]
  wv = wsc[...]
  h = h + xv[:, 0:1] * wv[0:1, :] + xv[:, 1:2] * wv[1:2, :] + b[...]
  h_ref[...] = h

  @pl.when(pl.program_id(0) == 0)
  def _():
    s_ref[...] = jnp.zeros_like(s_ref)
    q_ref[...] = jnp.zeros_like(q_ref)

  s_ref[...] += jnp.sum(h, axis=0, keepdims=True)
  q_ref[...] += jnp.sum(h * h, axis=0, keepdims=True)


def _l2_body(h0, sc, sh, w, b, h_ref, s_ref, q_ref):
  a = jnp.maximum(h0[...] * sc[...] + sh[...], 0.0)
  h = jnp.dot(a, w[...], preferred_element_type=jnp.float32,
              precision=lax.Precision.HIGHEST) + b[...]
  h_ref[...] = h

  @pl.when(pl.program_id(0) == 0)
  def _():
    s_ref[...] = jnp.zeros_like(s_ref)
    q_ref[...] = jnp.zeros_like(q_ref)

  s_ref[...] += jnp.sum(h, axis=0, keepdims=True)
  q_ref[...] += jnp.sum(h * h, axis=0, keepdims=True)


def _l3_body(h1, sc, sh, w, b, o_ref):
  a = jnp.maximum(h1[...] * sc[...] + sh[...], 0.0)
  o_ref[...] = jnp.dot(a, w[...], preferred_element_type=jnp.float32,
                       precision=lax.Precision.HIGHEST) + b[...]


def _full(shape):
  return pl.BlockSpec(shape, lambda i: (0, 0))


def _tile(width):
  return pl.BlockSpec((BT, width), lambda i: (i, 0))


def _layer1(feats, xsc, w, wsc, b):
  return pl.pallas_call(
      _l1_body,
      grid=(B // BT,),
      in_specs=[_tile(E // 2)] * 5 + [_tile(E)] + [_tile(2), _full((6 * E, 2 * D1)),
                                 _full((2, 2 * D1)), _full((1, 2 * D1))],
      out_specs=[_tile(2 * D1), _full((1, 2 * D1)), _full((1, 2 * D1))],
      out_shape=[jax.ShapeDtypeStruct((B, 2 * D1), jnp.float32),
                 jax.ShapeDtypeStruct((1, 2 * D1), jnp.float32),
                 jax.ShapeDtypeStruct((1, 2 * D1), jnp.float32)],
  )(*feats, xsc, w, wsc, b)


def _layer2(h0, sc, sh, w, b):
  return pl.pallas_call(
      _l2_body,
      grid=(B // BT,),
      in_specs=[_tile(2 * D1), _full((1, 2 * D1)), _full((1, 2 * D1)),
                _full((2 * D1, 2 * D2)), _full((1, 2 * D2))],
      out_specs=[_tile(2 * D2), _full((1, 2 * D2)), _full((1, 2 * D2))],
      out_shape=[jax.ShapeDtypeStruct((B, 2 * D2), jnp.float32),
                 jax.ShapeDtypeStruct((1, 2 * D2), jnp.float32),
                 jax.ShapeDtypeStruct((1, 2 * D2), jnp.float32)],
  )(h0, sc, sh, w, b)


def _layer3(h1, sc, sh, w, b):
  return pl.pallas_call(
      _l3_body,
      grid=(B // BT,),
      in_specs=[_tile(2 * D2), _full((1, 2 * D2)), _full((1, 2 * D2)),
                _full((2 * D2, 2)), _full((1, 2))],
      out_specs=_tile(2),
      out_shape=jax.ShapeDtypeStruct((B, 2), jnp.float32),
  )(h1, sc, sh, w, b)


def _bn_fold(s, q, g, be):
  mu = s / B
  var = q / B - mu * mu
  scale = g / jnp.sqrt(var + 1e-5)
  return scale, be - mu * scale


def kernel(x, x_seq, emb_user_id, emb_user_gender, emb_user_city, emb_item_id,
           emb_item_cate,
           t0_W0, t0_b0, t0_g0, t0_be0, t0_W1, t0_b1, t0_g1, t0_be1,
           t0_Wout, t0_bout,
           t1_W0, t1_b0, t1_g0, t1_be0, t1_W1, t1_b1, t1_g1, t1_be1,
           t1_Wout, t1_bout):
  xi = x.astype(jnp.int32)
  xseq_p = jnp.pad(x_seq.astype(jnp.int32), ((0, 0), (0, LP - L)))
  def _pack(t):
    u = lax.bitcast_convert_type(t, jnp.uint32)
    rb = (u + jnp.uint32(0x7FFF) + ((u >> 16) & jnp.uint32(1))) >> 16
    word = rb[:, :E // 2] | (rb[:, E // 2:] << 16)
    return lax.bitcast_convert_type(word, jnp.int32)

  tb_uid = _pack(emb_user_id)
  tb_gen = _pack(emb_user_gender)
  tb_city = _pack(emb_user_city)
  tb_iid = _pack(emb_item_id)
  tb_cate = _pack(emb_item_cate)
  feats = _sc_embed(xseq_p, xi[:, 0], xi[:, 2], xi[:, 3], xi[:, 4], xi[:, 5],
                    tb_uid, tb_gen, tb_city, tb_iid, tb_cate)
  xsc = jnp.stack([x[:, 1], x[:, 6]], axis=1)

  perm = jnp.array(_UNPACK_PERM, jnp.int32)

  def _permW(W):
    We = jnp.concatenate([W[0:64], W[65:129], W[129:193], W[193:257],
                          W[257:321], W[322:386][perm]], axis=0)
    return We, jnp.stack([W[64], W[321]], axis=0)

  W0e0, Wsc0 = _permW(t0_W0)
  W0e1, Wsc1 = _permW(t1_W0)
  W0cat = jnp.concatenate([W0e0, W0e1], axis=1)
  Wsccat = jnp.concatenate([Wsc0, Wsc1], axis=1)
  b0cat = jnp.concatenate([t0_b0, t1_b0])[None, :]
  g0cat = jnp.concatenate([t0_g0, t1_g0])[None, :]
  be0cat = jnp.concatenate([t0_be0, t1_be0])[None, :]
  W1bd = (jnp.zeros((2 * D1, 2 * D2), jnp.float32)
          .at[:D1, :D2].set(t0_W1).at[D1:, D2:].set(t1_W1))
  b1cat = jnp.concatenate([t0_b1, t1_b1])[None, :]
  g1cat = jnp.concatenate([t0_g1, t1_g1])[None, :]
  be1cat = jnp.concatenate([t0_be1, t1_be1])[None, :]
  Woutbd = (jnp.zeros((2 * D2, 2), jnp.float32)
            .at[:D2, 0:1].set(t0_Wout).at[D2:, 1:2].set(t1_Wout))
  boutcat = jnp.concatenate([t0_bout, t1_bout])[None, :]

  h0, s0, q0 = _layer1(feats, xsc, W0cat, Wsccat, b0cat)
  sc0, sh0 = _bn_fold(s0, q0, g0cat, be0cat)
  h1, s1, q1 = _layer2(h0, sc0, sh0, W1bd, b1cat)
  sc1, sh1 = _bn_fold(s1, q1, g1cat, be1cat)
  out = _layer3(h1, sc1, sh1, Woutbd, boutcat)
  return (out[:, 0:1], out[:, 1:2])


# speedup vs baseline: 1.9287x; 1.2031x over previous
import functools

import jax
import jax.numpy as jnp
from jax import lax
from jax.experimental import pallas as pl
from jax.experimental.pallas import tpu as pltpu
from jax.experimental.pallas import tpu_sc as plsc

B = 16384
L = 200
LP = 208
E = 64
NC = 2
NS = 16
NW = NC * NS
RPW = B // NW
D1, D2 = 256, 128
BT = 1024

BR = 128
NSLOT = 4

_DN = lax.GatherDimensionNumbers(offset_dims=(), collapsed_slice_dims=(0,),
                                 start_index_map=(0,))

_UNPACK_PERM = (list(range(0, 16)) + list(range(32, 48))
                + list(range(16, 32)) + list(range(48, 64)))


def _bf16_unpack(wi):
  even = lax.bitcast_convert_type(lax.shift_left(wi, 16), jnp.float32)
  odd = lax.bitcast_convert_type(jnp.bitwise_and(wi, jnp.int32(-65536)),
                                 jnp.float32)
  return even, odd


def _lane_total(v):
  lanes = lax.iota(jnp.int32, 16)
  for shift in (8, 4, 2, 1):
    perm = jnp.bitwise_xor(lanes, shift)
    v = v + lax.gather(v, perm[:, None], _DN, (1,),
                       mode=lax.GatherScatterMode.PROMISE_IN_BOUNDS)
  return v


def _sc_features(xseq, i_uid, i_gen, i_city, i_iid, i_cate,
                 t_uid, t_gen, t_city, t_iid, t_cate, t_seq,
                 o_uid, o_gen, o_city, o_iid, o_cate, o_seq,
                 idxf_v, fbuf_v, idxblk_v, rows_v, sbuf_v, e0_v,
                 semf, sem0, sem1, sem2, sem3):
  wid = lax.axis_index("s") * NC + lax.axis_index("c")
  base = wid * RPW

  for idx_hbm, tab, out in ((i_uid, t_uid, o_uid), (i_gen, t_gen, o_gen),
                            (i_city, t_city, o_city), (i_iid, t_iid, o_iid),
                            (i_cate, t_cate, o_cate)):
    pltpu.sync_copy(idx_hbm.at[pl.ds(base, RPW)], idxf_v)
    for c in range(RPW // 128):
      pltpu.make_async_copy(tab.at[idxf_v.at[pl.ds(c * 128, 128)]],
                            fbuf_v.at[pl.ds(c * 128, 128)], semf).start()
    for c in range(RPW // 128):
      pltpu.make_async_copy(tab.at[idxf_v.at[pl.ds(c * 128, 128)]],
                            fbuf_v.at[pl.ds(c * 128, 128)], semf).wait()
    pltpu.sync_copy(fbuf_v, out.at[pl.ds(base, RPW)])

  pltpu.sync_copy(t_seq.at[0], e0_v)
  e0a0, e0b0 = _bf16_unpack(e0_v[pl.ds(0, 16)])
  e0a1, e0b1 = _bf16_unpack(e0_v[pl.ds(16, 16)])
  e0ch = (e0a0, e0b0, e0a1, e0b1)
  sems = (sem0, sem1, sem2, sem3)

  def _gathers(r, s):
    return ((t_seq.at[idxblk_v.at[r, pl.ds(0, 128)]],
             sbuf_v.at[s, pl.ds(0, 128)], sems[s]),
            (t_seq.at[idxblk_v.at[r, pl.ds(128, LP - 128)]],
             sbuf_v.at[s, pl.ds(128, LP - 128)], sems[s]))

  def issue(r, s):
    for src, dst, sem in _gathers(r, s):
      pltpu.make_async_copy(src, dst, sem).start()

  def drain(r, s):
    for src, dst, sem in _gathers(r, s):
      pltpu.make_async_copy(src, dst, sem).wait()

  def compute(r, s):
    nzi = jnp.zeros((16,), jnp.int32)
    for c in range(LP // 16):
      ch = idxblk_v[r, pl.ds(c * 16, 16)]
      nzi = nzi + (1 - jnp.minimum(ch, 1))
    n0v = _lane_total(nzi.astype(jnp.float32))
    rcp = jnp.float32(1.0) / (jnp.float32(LP) - n0v + jnp.float32(1e-8))

    def sbody(j, accs):
      out = list(accs)
      for u in range(8):
        row = j * 8 + u
        for h in range(2):
          w = sbuf_v[s, row, pl.ds(h * 16, 16)]
          a, b = _bf16_unpack(w)
          out[2 * h] = out[2 * h] + a
          out[2 * h + 1] = out[2 * h + 1] + b
      return tuple(out)

    accs = lax.fori_loop(0, LP // 8, sbody,
                         tuple(jnp.zeros((16,), jnp.float32) for _ in range(4)))
    for c in range(4):
      avg = (accs[c] - n0v * e0ch[c]) * rcp
      rows_v[r, pl.ds(c * 16, 16)] = avg

  for blk in range(RPW // BR):
    pltpu.sync_copy(xseq.at[pl.ds(base + blk * BR, BR)], idxblk_v)
    for s in range(NSLOT):
      issue(s, s)

    def group(g, carry):
      r = g * NSLOT
      for s in range(NSLOT):
        drain(r + s, s)
        compute(r + s, s)

        @pl.when(r + s + NSLOT < BR)
        def _():
          issue(r + s + NSLOT, s)

      return carry

    lax.fori_loop(0, BR // NSLOT, group, jnp.int32(0))
    pltpu.sync_copy(rows_v, o_seq.at[pl.ds(base + blk * BR, BR)])


_sc_embed = functools.partial(
    pl.kernel,
    out_type=[jax.ShapeDtypeStruct((B, E), jnp.float32)] * 6,
    mesh=plsc.VectorSubcoreMesh(core_axis_name="c", subcore_axis_name="s"),
    compiler_params=pltpu.CompilerParams(use_tc_tiling_on_sc=False),
    scratch_types=[
        pltpu.VMEM((RPW,), jnp.int32),
        pltpu.VMEM((RPW, E), jnp.float32),
        pltpu.VMEM((BR, LP), jnp.int32),
        pltpu.VMEM((BR, E), jnp.float32),
        pltpu.VMEM((NSLOT, LP, E // 2), jnp.int32),
        pltpu.VMEM((E // 2,), jnp.int32),
        pltpu.SemaphoreType.DMA,
        pltpu.SemaphoreType.DMA,
        pltpu.SemaphoreType.DMA,
        pltpu.SemaphoreType.DMA,
        pltpu.SemaphoreType.DMA,
    ],
)(_sc_features)


def _l1_body(f0, f1, f2, f3, f4, f5, xsc, w, wsc, b, h_ref, s_ref, q_ref):
  hid = jnp.concatenate(
      [f0[...], f1[...], f2[...], f3[...], f4[...], f5[...]], axis=1)
  h = jnp.dot(hid, w[...], preferred_element_type=jnp.float32,
              precision=lax.Precision.HIGHEST)
  xv = xsc[...]
  wv = wsc[...]
  h = h + xv[:, 0:1] * wv[0:1, :] + xv[:, 1:2] * wv[1:2, :] + b[...]
  h_ref[...] = h

  @pl.when(pl.program_id(0) == 0)
  def _():
    s_ref[...] = jnp.zeros_like(s_ref)
    q_ref[...] = jnp.zeros_like(q_ref)

  s_ref[...] += jnp.sum(h, axis=0, keepdims=True)
  q_ref[...] += jnp.sum(h * h, axis=0, keepdims=True)


def _l2_body(h0, sc, sh, w, b, h_ref, s_ref, q_ref):
  a = jnp.maximum(h0[...] * sc[...] + sh[...], 0.0)
  h = jnp.dot(a, w[...], preferred_element_type=jnp.float32,
              precision=lax.Precision.HIGHEST) + b[...]
  h_ref[...] = h

  @pl.when(pl.program_id(0) == 0)
  def _():
    s_ref[...] = jnp.zeros_like(s_ref)
    q_ref[...] = jnp.zeros_like(q_ref)

  s_ref[...] += jnp.sum(h, axis=0, keepdims=True)
  q_ref[...] += jnp.sum(h * h, axis=0, keepdims=True)


def _l3_body(h1, sc, sh, w, b, o_ref):
  a = jnp.maximum(h1[...] * sc[...] + sh[...], 0.0)
  o_ref[...] = jnp.dot(a, w[...], preferred_element_type=jnp.float32,
                       precision=lax.Precision.HIGHEST) + b[...]


def _full(shape):
  return pl.BlockSpec(shape, lambda i: (0, 0))


def _tile(width):
  return pl.BlockSpec((BT, width), lambda i: (i, 0))


def _layer1(feats, xsc, w, wsc, b):
  return pl.pallas_call(
      _l1_body,
      grid=(B // BT,),
      in_specs=[_tile(E)] * 6 + [_tile(2), _full((6 * E, 2 * D1)),
                                 _full((2, 2 * D1)), _full((1, 2 * D1))],
      out_specs=[_tile(2 * D1), _full((1, 2 * D1)), _full((1, 2 * D1))],
      out_shape=[jax.ShapeDtypeStruct((B, 2 * D1), jnp.float32),
                 jax.ShapeDtypeStruct((1, 2 * D1), jnp.float32),
                 jax.ShapeDtypeStruct((1, 2 * D1), jnp.float32)],
  )(*feats, xsc, w, wsc, b)


def _layer2(h0, sc, sh, w, b):
  return pl.pallas_call(
      _l2_body,
      grid=(B // BT,),
      in_specs=[_tile(2 * D1), _full((1, 2 * D1)), _full((1, 2 * D1)),
                _full((2 * D1, 2 * D2)), _full((1, 2 * D2))],
      out_specs=[_tile(2 * D2), _full((1, 2 * D2)), _full((1, 2 * D2))],
      out_shape=[jax.ShapeDtypeStruct((B, 2 * D2), jnp.float32),
                 jax.ShapeDtypeStruct((1, 2 * D2), jnp.float32),
                 jax.ShapeDtypeStruct((1, 2 * D2), jnp.float32)],
  )(h0, sc, sh, w, b)


def _layer3(h1, sc, sh, w, b):
  return pl.pallas_call(
      _l3_body,
      grid=(B // BT,),
      in_specs=[_tile(2 * D2), _full((1, 2 * D2)), _full((1, 2 * D2)),
                _full((2 * D2, 2)), _full((1, 2))],
      out_specs=_tile(2),
      out_shape=jax.ShapeDtypeStruct((B, 2), jnp.float32),
  )(h1, sc, sh, w, b)


def _bn_fold(s, q, g, be):
  mu = s / B
  var = q / B - mu * mu
  scale = g / jnp.sqrt(var + 1e-5)
  return scale, be - mu * scale


def kernel(x, x_seq, emb_user_id, emb_user_gender, emb_user_city, emb_item_id,
           emb_item_cate,
           t0_W0, t0_b0, t0_g0, t0_be0, t0_W1, t0_b1, t0_g1, t0_be1,
           t0_Wout, t0_bout,
           t1_W0, t1_b0, t1_g0, t1_be0, t1_W1, t1_b1, t1_g1, t1_be1,
           t1_Wout, t1_bout):
  xi = x.astype(jnp.int32)
  xseq_p = jnp.pad(x_seq.astype(jnp.int32), ((0, 0), (0, LP - L)))
  def _pack(t):
    u = lax.bitcast_convert_type(t.T, jnp.uint32)
    rb = (u + jnp.uint32(0x7FFF) + ((u >> 16) & jnp.uint32(1))) >> 16
    word = rb[:E // 2, :] | (rb[E // 2:, :] << 16)
    return lax.bitcast_convert_type(word, jnp.int32).T

  tb_seq = _pack(emb_item_id)
  feats = _sc_embed(xseq_p, xi[:, 0], xi[:, 2], xi[:, 3], xi[:, 4], xi[:, 5],
                    emb_user_id, emb_user_gender, emb_user_city, emb_item_id,
                    emb_item_cate, tb_seq)
  xsc = jnp.stack([x[:, 1], x[:, 6]], axis=1)

  perm = jnp.array(_UNPACK_PERM, jnp.int32)

  def _permW(W):
    We = jnp.concatenate([W[0:64], W[65:129], W[129:193], W[193:257],
                          W[257:321], W[322:386][perm]], axis=0)
    return We, jnp.stack([W[64], W[321]], axis=0)

  W0e0, Wsc0 = _permW(t0_W0)
  W0e1, Wsc1 = _permW(t1_W0)
  W0cat = jnp.concatenate([W0e0, W0e1], axis=1)
  Wsccat = jnp.concatenate([Wsc0, Wsc1], axis=1)
  b0cat = jnp.concatenate([t0_b0, t1_b0])[None, :]
  g0cat = jnp.concatenate([t0_g0, t1_g0])[None, :]
  be0cat = jnp.concatenate([t0_be0, t1_be0])[None, :]
  W1bd = (jnp.zeros((2 * D1, 2 * D2), jnp.float32)
          .at[:D1, :D2].set(t0_W1).at[D1:, D2:].set(t1_W1))
  b1cat = jnp.concatenate([t0_b1, t1_b1])[None, :]
  g1cat = jnp.concatenate([t0_g1, t1_g1])[None, :]
  be1cat = jnp.concatenate([t0_be1, t1_be1])[None, :]
  Woutbd = (jnp.zeros((2 * D2, 2), jnp.float32)
            .at[:D2, 0:1].set(t0_Wout).at[D2:, 1:2].set(t1_Wout))
  boutcat = jnp.concatenate([t0_bout, t1_bout])[None, :]

  h0, s0, q0 = _layer1(feats, xsc, W0cat, Wsccat, b0cat)
  sc0, sh0 = _bn_fold(s0, q0, g0cat, be0cat)
  h1, s1, q1 = _layer2(h0, sc0, sh0, W1bd, b1cat)
  sc1, sh1 = _bn_fold(s1, q1, g1cat, be1cat)
  out = _layer3(h1, sc1, sh1, Woutbd, boutcat)
  return (out[:, 0:1], out[:, 1:2])


# speedup vs baseline: 1.9331x; 1.0023x over previous
import functools

import jax
import jax.numpy as jnp
from jax import lax
from jax.experimental import pallas as pl
from jax.experimental.pallas import tpu as pltpu
from jax.experimental.pallas import tpu_sc as plsc

B = 16384
L = 200
LP = 208
E = 64
NC = 2
NS = 16
NW = NC * NS
RPW = B // NW
D1, D2 = 256, 128
BT = 1024

BR = 128
NSLOT = 4

_DN = lax.GatherDimensionNumbers(offset_dims=(), collapsed_slice_dims=(0,),
                                 start_index_map=(0,))

_UNPACK_PERM = (list(range(0, 16)) + list(range(32, 48))
                + list(range(16, 32)) + list(range(48, 64)))


def _bf16_unpack(wi):
  even = lax.bitcast_convert_type(lax.shift_left(wi, 16), jnp.float32)
  odd = lax.bitcast_convert_type(jnp.bitwise_and(wi, jnp.int32(-65536)),
                                 jnp.float32)
  return even, odd


def _lane_total(v):
  lanes = lax.iota(jnp.int32, 16)
  for shift in (8, 4, 2, 1):
    perm = jnp.bitwise_xor(lanes, shift)
    v = v + lax.gather(v, perm[:, None], _DN, (1,),
                       mode=lax.GatherScatterMode.PROMISE_IN_BOUNDS)
  return v


def _sc_features(xseq, i_uid, i_gen, i_city, i_iid, i_cate,
                 t_uid, t_gen, t_city, t_cate, t_seq,
                 o_uid, o_gen, o_city, o_iid, o_cate, o_seq,
                 idxf_v, fbuf_v, ibuf_v, idxblk_v, rows_v, sbuf_v, e0_v,
                 semf, sem0, sem1, sem2, sem3):
  wid = lax.axis_index("s") * NC + lax.axis_index("c")
  base = wid * RPW

  for idx_hbm, tab, buf, out in (
      (i_uid, t_uid, fbuf_v, o_uid), (i_gen, t_gen, fbuf_v, o_gen),
      (i_city, t_city, fbuf_v, o_city), (i_iid, t_seq, ibuf_v, o_iid),
      (i_cate, t_cate, fbuf_v, o_cate)):
    pltpu.sync_copy(idx_hbm.at[pl.ds(base, RPW)], idxf_v)
    for c in range(RPW // 128):
      pltpu.make_async_copy(tab.at[idxf_v.at[pl.ds(c * 128, 128)]],
                            buf.at[pl.ds(c * 128, 128)], semf).start()
    for c in range(RPW // 128):
      pltpu.make_async_copy(tab.at[idxf_v.at[pl.ds(c * 128, 128)]],
                            buf.at[pl.ds(c * 128, 128)], semf).wait()
    pltpu.sync_copy(buf, out.at[pl.ds(base, RPW)])

  pltpu.sync_copy(t_seq.at[0], e0_v)
  e0a0, e0b0 = _bf16_unpack(e0_v[pl.ds(0, 16)])
  e0a1, e0b1 = _bf16_unpack(e0_v[pl.ds(16, 16)])
  e0ch = (e0a0, e0b0, e0a1, e0b1)
  sems = (sem0, sem1, sem2, sem3)

  def _gathers(r, s):
    return ((t_seq.at[idxblk_v.at[r, pl.ds(0, 128)]],
             sbuf_v.at[s, pl.ds(0, 128)], sems[s]),
            (t_seq.at[idxblk_v.at[r, pl.ds(128, LP - 128)]],
             sbuf_v.at[s, pl.ds(128, LP - 128)], sems[s]))

  def issue(r, s):
    for src, dst, sem in _gathers(r, s):
      pltpu.make_async_copy(src, dst, sem).start()

  def drain(r, s):
    for src, dst, sem in _gathers(r, s):
      pltpu.make_async_copy(src, dst, sem).wait()

  def compute(r, s):
    nzi = jnp.zeros((16,), jnp.int32)
    for c in range(LP // 16):
      ch = idxblk_v[r, pl.ds(c * 16, 16)]
      nzi = nzi + (1 - jnp.minimum(ch, 1))
    n0v = _lane_total(nzi.astype(jnp.float32))
    rcp = jnp.float32(1.0) / (jnp.float32(LP) - n0v + jnp.float32(1e-8))

    def sbody(j, accs):
      out = list(accs)
      for u in range(8):
        row = j * 8 + u
        for h in range(2):
          w = sbuf_v[s, row, pl.ds(h * 16, 16)]
          a, b = _bf16_unpack(w)
          out[2 * h] = out[2 * h] + a
          out[2 * h + 1] = out[2 * h + 1] + b
      return tuple(out)

    accs = lax.fori_loop(0, LP // 8, sbody,
                         tuple(jnp.zeros((16,), jnp.float32) for _ in range(4)))
    for c in range(4):
      avg = (accs[c] - n0v * e0ch[c]) * rcp
      rows_v[r, pl.ds(c * 16, 16)] = avg

  for blk in range(RPW // BR):
    pltpu.sync_copy(xseq.at[pl.ds(base + blk * BR, BR)], idxblk_v)
    for s in range(NSLOT):
      issue(s, s)

    def group(g, carry):
      r = g * NSLOT
      for s in range(NSLOT):
        drain(r + s, s)
        compute(r + s, s)

        @pl.when(r + s + NSLOT < BR)
        def _():
          issue(r + s + NSLOT, s)

      return carry

    lax.fori_loop(0, BR // NSLOT, group, jnp.int32(0))
    pltpu.sync_copy(rows_v, o_seq.at[pl.ds(base + blk * BR, BR)])


_sc_embed = functools.partial(
    pl.kernel,
    out_type=[jax.ShapeDtypeStruct((B, E), jnp.float32)] * 3
    + [jax.ShapeDtypeStruct((B, E // 2), jnp.int32)]
    + [jax.ShapeDtypeStruct((B, E), jnp.float32)] * 2,
    mesh=plsc.VectorSubcoreMesh(core_axis_name="c", subcore_axis_name="s"),
    compiler_params=pltpu.CompilerParams(use_tc_tiling_on_sc=False),
    scratch_types=[
        pltpu.VMEM((RPW,), jnp.int32),
        pltpu.VMEM((RPW, E), jnp.float32),
        pltpu.VMEM((RPW, E // 2), jnp.int32),
        pltpu.VMEM((BR, LP), jnp.int32),
        pltpu.VMEM((BR, E), jnp.float32),
        pltpu.VMEM((NSLOT, LP, E // 2), jnp.int32),
        pltpu.VMEM((E // 2,), jnp.int32),
        pltpu.SemaphoreType.DMA,
        pltpu.SemaphoreType.DMA,
        pltpu.SemaphoreType.DMA,
        pltpu.SemaphoreType.DMA,
        pltpu.SemaphoreType.DMA,
    ],
)(_sc_features)


def _l1_body(f0, f1, f2, f3, f4, f5, xsc, w, wsc, b, h_ref, s_ref, q_ref):
  wi = f3[...]
  hid = jnp.concatenate(
      [f0[...], f1[...], f2[...],
       lax.bitcast_convert_type(wi << 16, jnp.float32),
       lax.bitcast_convert_type(wi & jnp.int32(-65536), jnp.float32),
       f4[...], f5[...]], axis=1)
  h = jnp.dot(hid, w[...], preferred_element_type=jnp.float32,
              precision=lax.Precision.HIGHEST)
  xv = xsc[...]
  wv = wsc[...]
  h = h + xv[:, 0:1] * wv[0:1, :] + xv[:, 1:2] * wv[1:2, :] + b[...]
  h_ref[...] = h

  @pl.when(pl.program_id(0) == 0)
  def _():
    s_ref[...] = jnp.zeros_like(s_ref)
    q_ref[...] = jnp.zeros_like(q_ref)

  s_ref[...] += jnp.sum(h, axis=0, keepdims=True)
  q_ref[...] += jnp.sum(h * h, axis=0, keepdims=True)


def _l2_body(h0, sc, sh, w, b, h_ref, s_ref, q_ref):
  a = jnp.maximum(h0[...] * sc[...] + sh[...], 0.0)
  h = jnp.dot(a, w[...], preferred_element_type=jnp.float32,
              precision=lax.Precision.HIGHEST) + b[...]
  h_ref[...] = h

  @pl.when(pl.program_id(0) == 0)
  def _():
    s_ref[...] = jnp.zeros_like(s_ref)
    q_ref[...] = jnp.zeros_like(q_ref)

  s_ref[...] += jnp.sum(h, axis=0, keepdims=True)
  q_ref[...] += jnp.sum(h * h, axis=0, keepdims=True)


def _l3_body(h1, sc, sh, w, b, o_ref):
  a = jnp.maximum(h1[...] * sc[...] + sh[...], 0.0)
  o_ref[...] = jnp.dot(a, w[...], preferred_element_type=jnp.float32,
                       precision=lax.Precision.HIGHEST) + b[...]


def _full(shape):
  return pl.BlockSpec(shape, lambda i: (0, 0))


def _tile(width):
  return pl.BlockSpec((BT, width), lambda i: (i, 0))


def _layer1(feats, xsc, w, wsc, b):
  return pl.pallas_call(
      _l1_body,
      grid=(B // BT,),
      in_specs=[_tile(E)] * 3 + [_tile(E // 2)] + [_tile(E)] * 2
      + [_tile(2), _full((6 * E, 2 * D1)),
                                 _full((2, 2 * D1)), _full((1, 2 * D1))],
      out_specs=[_tile(2 * D1), _full((1, 2 * D1)), _full((1, 2 * D1))],
      out_shape=[jax.ShapeDtypeStruct((B, 2 * D1), jnp.float32),
                 jax.ShapeDtypeStruct((1, 2 * D1), jnp.float32),
                 jax.ShapeDtypeStruct((1, 2 * D1), jnp.float32)],
  )(*feats, xsc, w, wsc, b)


def _layer2(h0, sc, sh, w, b):
  return pl.pallas_call(
      _l2_body,
      grid=(B // BT,),
      in_specs=[_tile(2 * D1), _full((1, 2 * D1)), _full((1, 2 * D1)),
                _full((2 * D1, 2 * D2)), _full((1, 2 * D2))],
      out_specs=[_tile(2 * D2), _full((1, 2 * D2)), _full((1, 2 * D2))],
      out_shape=[jax.ShapeDtypeStruct((B, 2 * D2), jnp.float32),
                 jax.ShapeDtypeStruct((1, 2 * D2), jnp.float32),
                 jax.ShapeDtypeStruct((1, 2 * D2), jnp.float32)],
  )(h0, sc, sh, w, b)


def _layer3(h1, sc, sh, w, b):
  return pl.pallas_call(
      _l3_body,
      grid=(B // BT,),
      in_specs=[_tile(2 * D2), _full((1, 2 * D2)), _full((1, 2 * D2)),
                _full((2 * D2, 2)), _full((1, 2))],
      out_specs=_tile(2),
      out_shape=jax.ShapeDtypeStruct((B, 2), jnp.float32),
  )(h1, sc, sh, w, b)


def _bn_fold(s, q, g, be):
  mu = s / B
  var = q / B - mu * mu
  scale = g / jnp.sqrt(var + 1e-5)
  return scale, be - mu * scale


def kernel(x, x_seq, emb_user_id, emb_user_gender, emb_user_city, emb_item_id,
           emb_item_cate,
           t0_W0, t0_b0, t0_g0, t0_be0, t0_W1, t0_b1, t0_g1, t0_be1,
           t0_Wout, t0_bout,
           t1_W0, t1_b0, t1_g0, t1_be0, t1_W1, t1_b1, t1_g1, t1_be1,
           t1_Wout, t1_bout):
  xi = x.astype(jnp.int32)
  xseq_p = jnp.pad(x_seq.astype(jnp.int32), ((0, 0), (0, LP - L)))
  def _pack(t):
    u = lax.bitcast_convert_type(t.T, jnp.uint32)
    rb = (u + jnp.uint32(0x7FFF) + ((u >> 16) & jnp.uint32(1))) >> 16
    word = rb[:E // 2, :] | (rb[E // 2:, :] << 16)
    return lax.bitcast_convert_type(word, jnp.int32).T

  tb_seq = _pack(emb_item_id)
  feats = _sc_embed(xseq_p, xi[:, 0], xi[:, 2], xi[:, 3], xi[:, 4], xi[:, 5],
                    emb_user_id, emb_user_gender, emb_user_city,
                    emb_item_cate, tb_seq)
  xsc = jnp.stack([x[:, 1], x[:, 6]], axis=1)

  perm = jnp.array(_UNPACK_PERM, jnp.int32)

  def _permW(W):
    We = jnp.concatenate([W[0:64], W[65:129], W[129:193], W[193:257],
                          W[257:321], W[322:386][perm]], axis=0)
    return We, jnp.stack([W[64], W[321]], axis=0)

  W0e0, Wsc0 = _permW(t0_W0)
  W0e1, Wsc1 = _permW(t1_W0)
  W0cat = jnp.concatenate([W0e0, W0e1], axis=1)
  Wsccat = jnp.concatenate([Wsc0, Wsc1], axis=1)
  b0cat = jnp.concatenate([t0_b0, t1_b0])[None, :]
  g0cat = jnp.concatenate([t0_g0, t1_g0])[None, :]
  be0cat = jnp.concatenate([t0_be0, t1_be0])[None, :]
  W1bd = (jnp.zeros((2 * D1, 2 * D2), jnp.float32)
          .at[:D1, :D2].set(t0_W1).at[D1:, D2:].set(t1_W1))
  b1cat = jnp.concatenate([t0_b1, t1_b1])[None, :]
  g1cat = jnp.concatenate([t0_g1, t1_g1])[None, :]
  be1cat = jnp.concatenate([t0_be1, t1_be1])[None, :]
  Woutbd = (jnp.zeros((2 * D2, 2), jnp.float32)
            .at[:D2, 0:1].set(t0_Wout).at[D2:, 1:2].set(t1_Wout))
  boutcat = jnp.concatenate([t0_bout, t1_bout])[None, :]

  h0, s0, q0 = _layer1(feats, xsc, W0cat, Wsccat, b0cat)
  sc0, sh0 = _bn_fold(s0, q0, g0cat, be0cat)
  h1, s1, q1 = _layer2(h0, sc0, sh0, W1bd, b1cat)
  sc1, sh1 = _bn_fold(s1, q1, g1cat, be1cat)
  out = _layer3(h1, sc1, sh1, Woutbd, boutcat)
  return (out[:, 0:1], out[:, 1:2])


# speedup vs baseline: 1.9667x; 1.0174x over previous
import functools

import jax
import jax.numpy as jnp
from jax import lax
from jax.experimental import pallas as pl
from jax.experimental.pallas import tpu as pltpu
from jax.experimental.pallas import tpu_sc as plsc

B = 16384
L = 200
LP = 208
E = 64
NC = 2
NS = 16
NW = NC * NS
RPW = B // NW
D1, D2 = 256, 128
BT = 1024

BR = 128
NSLOT = 4

_DN = lax.GatherDimensionNumbers(offset_dims=(), collapsed_slice_dims=(0,),
                                 start_index_map=(0,))

_UNPACK_PERM = (list(range(0, 16)) + list(range(32, 48))
                + list(range(16, 32)) + list(range(48, 64)))


def _bf16_unpack(wi):
  even = lax.bitcast_convert_type(lax.shift_left(wi, 16), jnp.float32)
  odd = lax.bitcast_convert_type(jnp.bitwise_and(wi, jnp.int32(-65536)),
                                 jnp.float32)
  return even, odd


def _lane_total(v):
  lanes = lax.iota(jnp.int32, 16)
  for shift in (8, 4, 2, 1):
    perm = jnp.bitwise_xor(lanes, shift)
    v = v + lax.gather(v, perm[:, None], _DN, (1,),
                       mode=lax.GatherScatterMode.PROMISE_IN_BOUNDS)
  return v


def _sc_feats(i_uid, i_gen, i_city, i_cate,
              t_uid, t_gen, t_city, t_cate,
              o_uid, o_gen, o_city, o_cate,
              idxf_v, fbuf_v, semf):
  wid = lax.axis_index("s") * NC + lax.axis_index("c")
  base = wid * RPW

  for idx_hbm, tab, out in ((i_uid, t_uid, o_uid), (i_gen, t_gen, o_gen),
                            (i_city, t_city, o_city), (i_cate, t_cate, o_cate)):
    pltpu.sync_copy(idx_hbm.at[pl.ds(base, RPW)], idxf_v)
    for c in range(RPW // 128):
      pltpu.make_async_copy(tab.at[idxf_v.at[pl.ds(c * 128, 128)]],
                            fbuf_v.at[pl.ds(c * 128, 128)], semf).start()
    for c in range(RPW // 128):
      pltpu.make_async_copy(tab.at[idxf_v.at[pl.ds(c * 128, 128)]],
                            fbuf_v.at[pl.ds(c * 128, 128)], semf).wait()
    pltpu.sync_copy(fbuf_v, out.at[pl.ds(base, RPW)])


_sc_feats_call = functools.partial(
    pl.kernel,
    out_type=[jax.ShapeDtypeStruct((B, E), jnp.float32)] * 4,
    mesh=plsc.VectorSubcoreMesh(core_axis_name="c", subcore_axis_name="s"),
    compiler_params=pltpu.CompilerParams(use_tc_tiling_on_sc=False),
    scratch_types=[
        pltpu.VMEM((RPW,), jnp.int32),
        pltpu.VMEM((RPW, E), jnp.float32),
        pltpu.SemaphoreType.DMA,
    ],
)(_sc_feats)


def _sc_seq(xseq, i_iid, t_seq, o_iid, o_seq,
            idxf_v, ibuf_v, idxblk_v, rows_v, sbuf_v, e0_v,
            semf, sem0, sem1, sem2, sem3):
  wid = lax.axis_index("s") * NC + lax.axis_index("c")
  base = wid * RPW

  pltpu.sync_copy(i_iid.at[pl.ds(base, RPW)], idxf_v)
  for c in range(RPW // 128):
    pltpu.make_async_copy(t_seq.at[idxf_v.at[pl.ds(c * 128, 128)]],
                          ibuf_v.at[pl.ds(c * 128, 128)], semf).start()
  for c in range(RPW // 128):
    pltpu.make_async_copy(t_seq.at[idxf_v.at[pl.ds(c * 128, 128)]],
                          ibuf_v.at[pl.ds(c * 128, 128)], semf).wait()
  pltpu.sync_copy(ibuf_v, o_iid.at[pl.ds(base, RPW)])

  pltpu.sync_copy(t_seq.at[0], e0_v)
  e0a0, e0b0 = _bf16_unpack(e0_v[pl.ds(0, 16)])
  e0a1, e0b1 = _bf16_unpack(e0_v[pl.ds(16, 16)])
  e0ch = (e0a0, e0b0, e0a1, e0b1)
  sems = (sem0, sem1, sem2, sem3)

  def _gathers(r, s):
    return ((t_seq.at[idxblk_v.at[r, pl.ds(0, 128)]],
             sbuf_v.at[s, pl.ds(0, 128)], sems[s]),
            (t_seq.at[idxblk_v.at[r, pl.ds(128, LP - 128)]],
             sbuf_v.at[s, pl.ds(128, LP - 128)], sems[s]))

  def issue(r, s):
    for src, dst, sem in _gathers(r, s):
      pltpu.make_async_copy(src, dst, sem).start()

  def drain(r, s):
    for src, dst, sem in _gathers(r, s):
      pltpu.make_async_copy(src, dst, sem).wait()

  def compute(r, s):
    nzi = jnp.zeros((16,), jnp.int32)
    for c in range(LP // 16):
      ch = idxblk_v[r, pl.ds(c * 16, 16)]
      nzi = nzi + (1 - jnp.minimum(ch, 1))
    n0v = _lane_total(nzi.astype(jnp.float32))
    rcp = jnp.float32(1.0) / (jnp.float32(LP) - n0v + jnp.float32(1e-8))

    def sbody(j, accs):
      out = list(accs)
      for u in range(8):
        row = j * 8 + u
        for h in range(2):
          w = sbuf_v[s, row, pl.ds(h * 16, 16)]
          a, b = _bf16_unpack(w)
          out[2 * h] = out[2 * h] + a
          out[2 * h + 1] = out[2 * h + 1] + b
      return tuple(out)

    accs = lax.fori_loop(0, LP // 8, sbody,
                         tuple(jnp.zeros((16,), jnp.float32) for _ in range(4)))
    for c in range(4):
      avg = (accs[c] - n0v * e0ch[c]) * rcp
      rows_v[r, pl.ds(c * 16, 16)] = avg

  for blk in range(RPW // BR):
    pltpu.sync_copy(xseq.at[pl.ds(base + blk * BR, BR)], idxblk_v)
    for s in range(NSLOT):
      issue(s, s)

    def group(g, carry):
      r = g * NSLOT
      for s in range(NSLOT):
        drain(r + s, s)
        compute(r + s, s)

        @pl.when(r + s + NSLOT < BR)
        def _():
          issue(r + s + NSLOT, s)

      return carry

    lax.fori_loop(0, BR // NSLOT, group, jnp.int32(0))
    pltpu.sync_copy(rows_v, o_seq.at[pl.ds(base + blk * BR, BR)])


_sc_seq_call = functools.partial(
    pl.kernel,
    out_type=[jax.ShapeDtypeStruct((B, E // 2), jnp.int32),
              jax.ShapeDtypeStruct((B, E), jnp.float32)],
    mesh=plsc.VectorSubcoreMesh(core_axis_name="c", subcore_axis_name="s"),
    compiler_params=pltpu.CompilerParams(use_tc_tiling_on_sc=False),
    scratch_types=[
        pltpu.VMEM((RPW,), jnp.int32),
        pltpu.VMEM((RPW, E // 2), jnp.int32),
        pltpu.VMEM((BR, LP), jnp.int32),
        pltpu.VMEM((BR, E), jnp.float32),
        pltpu.VMEM((NSLOT, LP, E // 2), jnp.int32),
        pltpu.VMEM((E // 2,), jnp.int32),
        pltpu.SemaphoreType.DMA,
        pltpu.SemaphoreType.DMA,
        pltpu.SemaphoreType.DMA,
        pltpu.SemaphoreType.DMA,
        pltpu.SemaphoreType.DMA,
    ],
)(_sc_seq)


def _l1_body(f0, f1, f2, f3, f4, f5, xsc, w, wsc, b, h_ref, s_ref, q_ref):
  wi = f3[...]
  hid = jnp.concatenate(
      [f0[...], f1[...], f2[...],
       lax.bitcast_convert_type(wi << 16, jnp.float32),
       lax.bitcast_convert_type(wi & jnp.int32(-65536), jnp.float32),
       f4[...], f5[...]], axis=1)
  h = jnp.dot(hid, w[...], preferred_element_type=jnp.float32,
              precision=lax.Precision.HIGHEST)
  xv = xsc[...]
  wv = wsc[...]
  h = h + xv[:, 0:1] * wv[0:1, :] + xv[:, 1:2] * wv[1:2, :] + b[...]
  h_ref[...] = h

  @pl.when(pl.program_id(0) == 0)
  def _():
    s_ref[...] = jnp.zeros_like(s_ref)
    q_ref[...] = jnp.zeros_like(q_ref)

  s_ref[...] += jnp.sum(h, axis=0, keepdims=True)
  q_ref[...] += jnp.sum(h * h, axis=0, keepdims=True)


def _l2_body(h0, sc, sh, w, b, h_ref, s_ref, q_ref):
  a = jnp.maximum(h0[...] * sc[...] + sh[...], 0.0)
  h = jnp.dot(a, w[...], preferred_element_type=jnp.float32,
              precision=lax.Precision.HIGHEST) + b[...]
  h_ref[...] = h

  @pl.when(pl.program_id(0) == 0)
  def _():
    s_ref[...] = jnp.zeros_like(s_ref)
    q_ref[...] = jnp.zeros_like(q_ref)

  s_ref[...] += jnp.sum(h, axis=0, keepdims=True)
  q_ref[...] += jnp.sum(h * h, axis=0, keepdims=True)


def _l3_body(h1, sc, sh, w, b, o_ref):
  a = jnp.maximum(h1[...] * sc[...] + sh[...], 0.0)
  o_ref[...] = jnp.dot(a, w[...], preferred_element_type=jnp.float32,
                       precision=lax.Precision.HIGHEST) + b[...]


def _full(shape):
  return pl.BlockSpec(shape, lambda i: (0, 0))


def _tile(width):
  return pl.BlockSpec((BT, width), lambda i: (i, 0))


def _layer1(feats, xsc, w, wsc, b):
  return pl.pallas_call(
      _l1_body,
      grid=(B // BT,),
      in_specs=[_tile(E)] * 3 + [_tile(E // 2)] + [_tile(E)] * 2
      + [_tile(2), _full((6 * E, 2 * D1)),
                                 _full((2, 2 * D1)), _full((1, 2 * D1))],
      out_specs=[_tile(2 * D1), _full((1, 2 * D1)), _full((1, 2 * D1))],
      out_shape=[jax.ShapeDtypeStruct((B, 2 * D1), jnp.float32),
                 jax.ShapeDtypeStruct((1, 2 * D1), jnp.float32),
                 jax.ShapeDtypeStruct((1, 2 * D1), jnp.float32)],
  )(*feats, xsc, w, wsc, b)


def _layer2(h0, sc, sh, w, b):
  return pl.pallas_call(
      _l2_body,
      grid=(B // BT,),
      in_specs=[_tile(2 * D1), _full((1, 2 * D1)), _full((1, 2 * D1)),
                _full((2 * D1, 2 * D2)), _full((1, 2 * D2))],
      out_specs=[_tile(2 * D2), _full((1, 2 * D2)), _full((1, 2 * D2))],
      out_shape=[jax.ShapeDtypeStruct((B, 2 * D2), jnp.float32),
                 jax.ShapeDtypeStruct((1, 2 * D2), jnp.float32),
                 jax.ShapeDtypeStruct((1, 2 * D2), jnp.float32)],
  )(h0, sc, sh, w, b)


def _layer3(h1, sc, sh, w, b):
  return pl.pallas_call(
      _l3_body,
      grid=(B // BT,),
      in_specs=[_tile(2 * D2), _full((1, 2 * D2)), _full((1, 2 * D2)),
                _full((2 * D2, 2)), _full((1, 2))],
      out_specs=_tile(2),
      out_shape=jax.ShapeDtypeStruct((B, 2), jnp.float32),
  )(h1, sc, sh, w, b)


def _bn_fold(s, q, g, be):
  mu = s / B
  var = q / B - mu * mu
  scale = g / jnp.sqrt(var + 1e-5)
  return scale, be - mu * scale


def kernel(x, x_seq, emb_user_id, emb_user_gender, emb_user_city, emb_item_id,
           emb_item_cate,
           t0_W0, t0_b0, t0_g0, t0_be0, t0_W1, t0_b1, t0_g1, t0_be1,
           t0_Wout, t0_bout,
           t1_W0, t1_b0, t1_g0, t1_be0, t1_W1, t1_b1, t1_g1, t1_be1,
           t1_Wout, t1_bout):
  xi = x.astype(jnp.int32)
  xseq_p = jnp.pad(x_seq.astype(jnp.int32), ((0, 0), (0, LP - L)))
  def _pack(t):
    u = lax.bitcast_convert_type(t.T, jnp.uint32)
    rb = (u + jnp.uint32(0x7FFF) + ((u >> 16) & jnp.uint32(1))) >> 16
    word = rb[:E // 2, :] | (rb[E // 2:, :] << 16)
    return lax.bitcast_convert_type(word, jnp.int32).T

  tb_seq = _pack(emb_item_id)
  f_uid, f_gen, f_city, f_cate = _sc_feats_call(
      xi[:, 0], xi[:, 2], xi[:, 3], xi[:, 5],
      emb_user_id, emb_user_gender, emb_user_city, emb_item_cate)
  f_iid, f_seq = _sc_seq_call(xseq_p, xi[:, 4], tb_seq)
  feats = (f_uid, f_gen, f_city, f_iid, f_cate, f_seq)
  xsc = jnp.stack([x[:, 1], x[:, 6]], axis=1)

  perm = jnp.array(_UNPACK_PERM, jnp.int32)

  def _permW(W):
    We = jnp.concatenate([W[0:64], W[65:129], W[129:193], W[193:257],
                          W[257:321], W[322:386][perm]], axis=0)
    return We, jnp.stack([W[64], W[321]], axis=0)

  W0e0, Wsc0 = _permW(t0_W0)
  W0e1, Wsc1 = _permW(t1_W0)
  W0cat = jnp.concatenate([W0e0, W0e1], axis=1)
  Wsccat = jnp.concatenate([Wsc0, Wsc1], axis=1)
  b0cat = jnp.concatenate([t0_b0, t1_b0])[None, :]
  g0cat = jnp.concatenate([t0_g0, t1_g0])[None, :]
  be0cat = jnp.concatenate([t0_be0, t1_be0])[None, :]
  W1bd = (jnp.zeros((2 * D1, 2 * D2), jnp.float32)
          .at[:D1, :D2].set(t0_W1).at[D1:, D2:].set(t1_W1))
  b1cat = jnp.concatenate([t0_b1, t1_b1])[None, :]
  g1cat = jnp.concatenate([t0_g1, t1_g1])[None, :]
  be1cat = jnp.concatenate([t0_be1, t1_be1])[None, :]
  Woutbd = (jnp.zeros((2 * D2, 2), jnp.float32)
            .at[:D2, 0:1].set(t0_Wout).at[D2:, 1:2].set(t1_Wout))
  boutcat = jnp.concatenate([t0_bout, t1_bout])[None, :]

  h0, s0, q0 = _layer1(feats, xsc, W0cat, Wsccat, b0cat)
  sc0, sh0 = _bn_fold(s0, q0, g0cat, be0cat)
  h1, s1, q1 = _layer2(h0, sc0, sh0, W1bd, b1cat)
  sc1, sh1 = _bn_fold(s1, q1, g1cat, be1cat)
  out = _layer3(h1, sc1, sh1, Woutbd, boutcat)
  return (out[:, 0:1], out[:, 1:2])


# speedup vs baseline: 3.0166x; 1.5338x over previous
import functools

import jax
import jax.numpy as jnp
from jax import lax
from jax.experimental import pallas as pl
from jax.experimental.pallas import tpu as pltpu
from jax.experimental.pallas import tpu_sc as plsc

B = 16384
L = 200
LP = 200
E = 64
NC = 2
NS = 16
NW = NC * NS
RPW = B // NW
D1, D2 = 256, 128
BT = 1024

BR = 128
NSLOT = 4

_DN = lax.GatherDimensionNumbers(offset_dims=(), collapsed_slice_dims=(0,),
                                 start_index_map=(0,))

_UNPACK_PERM = (list(range(0, 16)) + list(range(32, 48))
                + list(range(16, 32)) + list(range(48, 64)))


def _bf16_unpack(wi):
  even = lax.bitcast_convert_type(lax.shift_left(wi, 16), jnp.float32)
  odd = lax.bitcast_convert_type(jnp.bitwise_and(wi, jnp.int32(-65536)),
                                 jnp.float32)
  return even, odd


def _lane_total(v):
  lanes = lax.iota(jnp.int32, 16)
  for shift in (8, 4, 2, 1):
    perm = jnp.bitwise_xor(lanes, shift)
    v = v + lax.gather(v, perm[:, None], _DN, (1,),
                       mode=lax.GatherScatterMode.PROMISE_IN_BOUNDS)
  return v


def _sc_feats(i_uid, i_gen, i_city, i_cate,
              t_uid, t_gen, t_city, t_cate,
              o_uid, o_gen, o_city, o_cate,
              idxf_v, fbuf_v, semf):
  wid = lax.axis_index("s") * NC + lax.axis_index("c")
  base = wid * RPW

  for idx_hbm, tab, out in ((i_uid, t_uid, o_uid), (i_gen, t_gen, o_gen),
                            (i_city, t_city, o_city), (i_cate, t_cate, o_cate)):
    pltpu.sync_copy(idx_hbm.at[pl.ds(base, RPW)], idxf_v)
    for c in range(RPW // 128):
      pltpu.make_async_copy(tab.at[idxf_v.at[pl.ds(c * 128, 128)]],
                            fbuf_v.at[pl.ds(c * 128, 128)], semf).start()
    for c in range(RPW // 128):
      pltpu.make_async_copy(tab.at[idxf_v.at[pl.ds(c * 128, 128)]],
                            fbuf_v.at[pl.ds(c * 128, 128)], semf).wait()
    pltpu.sync_copy(fbuf_v, out.at[pl.ds(base, RPW)])


_sc_feats_call = functools.partial(
    pl.kernel,
    out_type=[jax.ShapeDtypeStruct((B, E), jnp.float32)] * 4,
    mesh=plsc.VectorSubcoreMesh(core_axis_name="c", subcore_axis_name="s"),
    compiler_params=pltpu.CompilerParams(use_tc_tiling_on_sc=False),
    scratch_types=[
        pltpu.VMEM((RPW,), jnp.int32),
        pltpu.VMEM((RPW, E), jnp.float32),
        pltpu.SemaphoreType.DMA,
    ],
)(_sc_feats)


def _sc_seq(xseq, i_iid, t_seq, o_iid, o_seq,
            idxf_v, ibuf_v, idxblk_v, rows_v, sbuf_v, e0_v,
            semf, sem0, sem1, sem2, sem3):
  wid = lax.axis_index("s") * NC + lax.axis_index("c")
  base = wid * RPW

  pltpu.sync_copy(i_iid.at[pl.ds(base, RPW)], idxf_v)
  for c in range(RPW // 128):
    pltpu.make_async_copy(t_seq.at[idxf_v.at[pl.ds(c * 128, 128)]],
                          ibuf_v.at[pl.ds(c * 128, 128)], semf).start()
  for c in range(RPW // 128):
    pltpu.make_async_copy(t_seq.at[idxf_v.at[pl.ds(c * 128, 128)]],
                          ibuf_v.at[pl.ds(c * 128, 128)], semf).wait()
  pltpu.sync_copy(ibuf_v, o_iid.at[pl.ds(base, RPW)])

  pltpu.sync_copy(t_seq.at[0], e0_v)
  e0a0, e0b0 = _bf16_unpack(e0_v[pl.ds(0, 16)])
  e0a1, e0b1 = _bf16_unpack(e0_v[pl.ds(16, 16)])
  e0ch = (e0a0, e0b0, e0a1, e0b1)
  sems = (sem0, sem1, sem2, sem3)

  def _gathers(r, s):
    return ((t_seq.at[idxblk_v.at[r, pl.ds(0, 128)]],
             sbuf_v.at[s, pl.ds(0, 128)], sems[s]),
            (t_seq.at[idxblk_v.at[r, pl.ds(128, LP - 128)]],
             sbuf_v.at[s, pl.ds(128, LP - 128)], sems[s]))

  def issue(r, s):
    for src, dst, sem in _gathers(r, s):
      pltpu.make_async_copy(src, dst, sem).start()

  def drain(r, s):
    for src, dst, sem in _gathers(r, s):
      pltpu.make_async_copy(src, dst, sem).wait()

  def compute(r, s):
    nzi = jnp.zeros((16,), jnp.int32)
    for c in range(LP // 16):
      ch = idxblk_v[r, pl.ds(c * 16, 16)]
      nzi = nzi + (1 - jnp.minimum(ch, 1))
    ch = idxblk_v[r, pl.ds(LP - 16, 16)]
    tail = lax.iota(jnp.int32, 16) >> 3
    nzi = nzi + (1 - jnp.minimum(ch, 1)) * tail
    n0v = _lane_total(nzi.astype(jnp.float32))
    rcp = jnp.float32(1.0) / (jnp.float32(LP) - n0v + jnp.float32(1e-8))

    def sbody(j, accs):
      out = list(accs)
      for u in range(8):
        row = j * 8 + u
        for h in range(2):
          w = sbuf_v[s, row, pl.ds(h * 16, 16)]
          a, b = _bf16_unpack(w)
          out[2 * h] = out[2 * h] + a
          out[2 * h + 1] = out[2 * h + 1] + b
      return tuple(out)

    accs = lax.fori_loop(0, LP // 8, sbody,
                         tuple(jnp.zeros((16,), jnp.float32) for _ in range(4)))
    for c in range(4):
      avg = (accs[c] - n0v * e0ch[c]) * rcp
      rows_v[r, pl.ds(c * 16, 16)] = avg

  for blk in range(RPW // BR):
    pltpu.sync_copy(xseq.at[pl.ds(base + blk * BR, BR)], idxblk_v)
    for s in range(NSLOT):
      issue(s, s)

    def group(g, carry):
      r = g * NSLOT
      for s in range(NSLOT):
        drain(r + s, s)
        compute(r + s, s)

        @pl.when(r + s + NSLOT < BR)
        def _():
          issue(r + s + NSLOT, s)

      return carry

    lax.fori_loop(0, BR // NSLOT, group, jnp.int32(0))
    pltpu.sync_copy(rows_v, o_seq.at[pl.ds(base + blk * BR, BR)])


_sc_seq_call = functools.partial(
    pl.kernel,
    out_type=[jax.ShapeDtypeStruct((B, E // 2), jnp.int32),
              jax.ShapeDtypeStruct((B, E), jnp.float32)],
    mesh=plsc.VectorSubcoreMesh(core_axis_name="c", subcore_axis_name="s"),
    compiler_params=pltpu.CompilerParams(use_tc_tiling_on_sc=False),
    scratch_types=[
        pltpu.VMEM((RPW,), jnp.int32),
        pltpu.VMEM((RPW, E // 2), jnp.int32),
        pltpu.VMEM((BR, LP), jnp.int32),
        pltpu.VMEM((BR, E), jnp.float32),
        pltpu.VMEM((NSLOT, LP, E // 2), jnp.int32),
        pltpu.VMEM((E // 2,), jnp.int32),
        pltpu.SemaphoreType.DMA,
        pltpu.SemaphoreType.DMA,
        pltpu.SemaphoreType.DMA,
        pltpu.SemaphoreType.DMA,
        pltpu.SemaphoreType.DMA,
    ],
)(_sc_seq)


def _l1_body(f0, f1, f2, f3, f4, f5, xsc, w, wsc, b, h_ref, s_ref, q_ref):
  wi = f3[...]
  hid = jnp.concatenate(
      [f0[...], f1[...], f2[...],
       lax.bitcast_convert_type(wi << 16, jnp.float32),
       lax.bitcast_convert_type(wi & jnp.int32(-65536), jnp.float32),
       f4[...], f5[...]], axis=1)
  h = jnp.dot(hid, w[...], preferred_element_type=jnp.float32,
              precision=lax.Precision.HIGHEST)
  xv = xsc[...]
  wv = wsc[...]
  h = h + xv[:, 0:1] * wv[0:1, :] + xv[:, 1:2] * wv[1:2, :] + b[...]
  h_ref[...] = h

  @pl.when(pl.program_id(0) == 0)
  def _():
    s_ref[...] = jnp.zeros_like(s_ref)
    q_ref[...] = jnp.zeros_like(q_ref)

  s_ref[...] += jnp.sum(h, axis=0, keepdims=True)
  q_ref[...] += jnp.sum(h * h, axis=0, keepdims=True)


def _l2_body(h0, sc, sh, w, b, h_ref, s_ref, q_ref):
  a = jnp.maximum(h0[...] * sc[...] + sh[...], 0.0)
  h = jnp.dot(a, w[...], preferred_element_type=jnp.float32,
              precision=lax.Precision.HIGHEST) + b[...]
  h_ref[...] = h

  @pl.when(pl.program_id(0) == 0)
  def _():
    s_ref[...] = jnp.zeros_like(s_ref)
    q_ref[...] = jnp.zeros_like(q_ref)

  s_ref[...] += jnp.sum(h, axis=0, keepdims=True)
  q_ref[...] += jnp.sum(h * h, axis=0, keepdims=True)


def _l3_body(h1, sc, sh, w, b, o_ref):
  a = jnp.maximum(h1[...] * sc[...] + sh[...], 0.0)
  o_ref[...] = jnp.dot(a, w[...], preferred_element_type=jnp.float32,
                       precision=lax.Precision.HIGHEST) + b[...]


def _full(shape):
  return pl.BlockSpec(shape, lambda i: (0, 0))


def _tile(width):
  return pl.BlockSpec((BT, width), lambda i: (i, 0))


def _layer1(feats, xsc, w, wsc, b):
  return pl.pallas_call(
      _l1_body,
      grid=(B // BT,),
      in_specs=[_tile(E)] * 3 + [_tile(E // 2)] + [_tile(E)] * 2
      + [_tile(2), _full((6 * E, 2 * D1)),
                                 _full((2, 2 * D1)), _full((1, 2 * D1))],
      out_specs=[_tile(2 * D1), _full((1, 2 * D1)), _full((1, 2 * D1))],
      out_shape=[jax.ShapeDtypeStruct((B, 2 * D1), jnp.float32),
                 jax.ShapeDtypeStruct((1, 2 * D1), jnp.float32),
                 jax.ShapeDtypeStruct((1, 2 * D1), jnp.float32)],
  )(*feats, xsc, w, wsc, b)


def _layer2(h0, sc, sh, w, b):
  return pl.pallas_call(
      _l2_body,
      grid=(B // BT,),
      in_specs=[_tile(2 * D1), _full((1, 2 * D1)), _full((1, 2 * D1)),
                _full((2 * D1, 2 * D2)), _full((1, 2 * D2))],
      out_specs=[_tile(2 * D2), _full((1, 2 * D2)), _full((1, 2 * D2))],
      out_shape=[jax.ShapeDtypeStruct((B, 2 * D2), jnp.float32),
                 jax.ShapeDtypeStruct((1, 2 * D2), jnp.float32),
                 jax.ShapeDtypeStruct((1, 2 * D2), jnp.float32)],
  )(h0, sc, sh, w, b)


def _layer3(h1, sc, sh, w, b):
  return pl.pallas_call(
      _l3_body,
      grid=(B // BT,),
      in_specs=[_tile(2 * D2), _full((1, 2 * D2)), _full((1, 2 * D2)),
                _full((2 * D2, 2)), _full((1, 2))],
      out_specs=_tile(2),
      out_shape=jax.ShapeDtypeStruct((B, 2), jnp.float32),
  )(h1, sc, sh, w, b)


def _bn_fold(s, q, g, be):
  mu = s / B
  var = q / B - mu * mu
  scale = g / jnp.sqrt(var + 1e-5)
  return scale, be - mu * scale


def kernel(x, x_seq, emb_user_id, emb_user_gender, emb_user_city, emb_item_id,
           emb_item_cate,
           t0_W0, t0_b0, t0_g0, t0_be0, t0_W1, t0_b1, t0_g1, t0_be1,
           t0_Wout, t0_bout,
           t1_W0, t1_b0, t1_g0, t1_be0, t1_W1, t1_b1, t1_g1, t1_be1,
           t1_Wout, t1_bout):
  xi = x.astype(jnp.int32)
  xseq_p = x_seq.astype(jnp.int32)
  def _pack(t):
    u = lax.bitcast_convert_type(t.T, jnp.uint32)
    rb = (u + jnp.uint32(0x7FFF) + ((u >> 16) & jnp.uint32(1))) >> 16
    word = rb[:E // 2, :] | (rb[E // 2:, :] << 16)
    return lax.bitcast_convert_type(word, jnp.int32).T

  tb_seq = _pack(emb_item_id)
  f_uid, f_gen, f_city, f_cate = _sc_feats_call(
      xi[:, 0], xi[:, 2], xi[:, 3], xi[:, 5],
      emb_user_id, emb_user_gender, emb_user_city, emb_item_cate)
  f_iid, f_seq = _sc_seq_call(xseq_p, xi[:, 4], tb_seq)
  feats = (f_uid, f_gen, f_city, f_iid, f_cate, f_seq)
  xsc = jnp.stack([x[:, 1], x[:, 6]], axis=1)

  perm = jnp.array(_UNPACK_PERM, jnp.int32)

  def _permW(W):
    We = jnp.concatenate([W[0:64], W[65:129], W[129:193], W[193:257],
                          W[257:321], W[322:386][perm]], axis=0)
    return We, jnp.stack([W[64], W[321]], axis=0)

  W0e0, Wsc0 = _permW(t0_W0)
  W0e1, Wsc1 = _permW(t1_W0)
  W0cat = jnp.concatenate([W0e0, W0e1], axis=1)
  Wsccat = jnp.concatenate([Wsc0, Wsc1], axis=1)
  b0cat = jnp.concatenate([t0_b0, t1_b0])[None, :]
  g0cat = jnp.concatenate([t0_g0, t1_g0])[None, :]
  be0cat = jnp.concatenate([t0_be0, t1_be0])[None, :]
  W1bd = (jnp.zeros((2 * D1, 2 * D2), jnp.float32)
          .at[:D1, :D2].set(t0_W1).at[D1:, D2:].set(t1_W1))
  b1cat = jnp.concatenate([t0_b1, t1_b1])[None, :]
  g1cat = jnp.concatenate([t0_g1, t1_g1])[None, :]
  be1cat = jnp.concatenate([t0_be1, t1_be1])[None, :]
  Woutbd = (jnp.zeros((2 * D2, 2), jnp.float32)
            .at[:D2, 0:1].set(t0_Wout).at[D2:, 1:2].set(t1_Wout))
  boutcat = jnp.concatenate([t0_bout, t1_bout])[None, :]

  h0, s0, q0 = _layer1(feats, xsc, W0cat, Wsccat, b0cat)
  sc0, sh0 = _bn_fold(s0, q0, g0cat, be0cat)
  h1, s1, q1 = _layer2(h0, sc0, sh0, W1bd, b1cat)
  sc1, sh1 = _bn_fold(s1, q1, g1cat, be1cat)
  out = _layer3(h1, sc1, sh1, Woutbd, boutcat)
  return (out[:, 0:1], out[:, 1:2])
